# Initial kernel scaffold; baseline (speedup 1.0000x reference)
import jax
import jax.numpy as jnp
from jax.experimental import pallas as pl


def kernel(x, edge_index, edge_features, params):
    raise NotImplementedError("write your pallas kernel here")


# speedup vs baseline: 3.0029x; 3.0029x over previous
import functools

import jax
import jax.numpy as jnp
from jax import lax
from jax.experimental import pallas as pl
from jax.experimental.pallas import tpu as pltpu
from jax.experimental.pallas import tpu_sc as plsc

N = 10000
E = 320000
D = 128

NC = 2
NS = 16
NW = NC * NS
EPW = E // NW
K = 80
NCHUNK = EPW // K
NP = 10240
RPS = NP // NS
ZR = 128

BN = 2000
BE = 2000

_MESH = plsc.VectorSubcoreMesh(core_axis_name="c", subcore_axis_name="s")


@functools.partial(
    pl.kernel,
    out_type=jax.ShapeDtypeStruct((E, D), jnp.float32),
    mesh=_MESH,
    scratch_types=[
        pltpu.VMEM((K,), jnp.int32),
        pltpu.VMEM((K,), jnp.int32),
        pltpu.VMEM((K, D), jnp.float32),
        pltpu.VMEM((K, D), jnp.float32),
        pltpu.SemaphoreType.DMA,
        pltpu.SemaphoreType.DMA,
    ],
)
def _sc_gather_uv(u_hbm, v_hbm, src_hbm, dst_hbm, out_hbm,
                  si_v, di_v, ub_v, vb_v, sem_u, sem_v):
    c = lax.axis_index("c")
    s = lax.axis_index("s")
    wid = s * NC + c
    base = wid * EPW

    @pl.loop(0, NCHUNK)
    def _chunk(ci):
        off = base + ci * K
        pltpu.sync_copy(src_hbm.at[pl.ds(off, K)], si_v)
        pltpu.sync_copy(dst_hbm.at[pl.ds(off, K)], di_v)
        cp_u = pltpu.async_copy(u_hbm.at[si_v], ub_v, sem_u)
        cp_v = pltpu.async_copy(v_hbm.at[di_v], vb_v, sem_v)
        cp_u.wait()
        cp_v.wait()

        @pl.loop(0, K)
        def _row(i):
            @pl.loop(0, D // 16, unroll=8)
            def _col(j):
                sl = pl.ds(j * 16, 16)
                ub_v[i, sl] = ub_v[i, sl] + vb_v[i, sl]

        pltpu.sync_copy(ub_v, out_hbm.at[pl.ds(off, K), :])


@functools.partial(
    pl.kernel,
    out_type=jax.ShapeDtypeStruct((2 * NP, D), jnp.float32),
    mesh=_MESH,
    scratch_types=[
        pltpu.VMEM((K,), jnp.int32),
        pltpu.VMEM((K, D), jnp.float32),
        pltpu.VMEM((ZR, D), jnp.float32),
        pltpu.VMEM_SHARED((NP, D), jnp.float32),
    ],
)
def _sc_scatter_add(e_hbm, dst_hbm, out_hbm, di_v, rows_v, zb_v, acc_sh):
    c = lax.axis_index("c")
    s = lax.axis_index("s")
    wid = s * NC + c
    base = wid * EPW

    @pl.loop(0, ZR)
    def _zrow(i):
        @pl.loop(0, D // 16, unroll=8)
        def _zcol(j):
            zb_v[i, pl.ds(j * 16, 16)] = jnp.zeros((16,), jnp.float32)

    @pl.loop(0, RPS // ZR)
    def _zcopy(k):
        pltpu.sync_copy(zb_v, acc_sh.at[pl.ds(s * RPS + k * ZR, ZR), :])

    plsc.subcore_barrier()

    @pl.loop(0, NCHUNK)
    def _chunk(ci):
        off = base + ci * K
        pltpu.sync_copy(e_hbm.at[pl.ds(off, K), :], rows_v)
        pltpu.sync_copy(dst_hbm.at[pl.ds(off, K)], di_v)
        pltpu.sync_copy(rows_v, acc_sh.at[di_v], add=True)

    plsc.subcore_barrier()
    pltpu.sync_copy(acc_sh.at[pl.ds(s * RPS, RPS), :],
                    out_hbm.at[pl.ds(c * NP + s * RPS, RPS), :])


def _proj_body(x_ref, wa_ref, wb_ref, u_ref, v_ref):
    x = x_ref[...]
    u_ref[...] = jnp.dot(x, wa_ref[...], preferred_element_type=jnp.float32)
    v_ref[...] = jnp.dot(x, wb_ref[...], preferred_element_type=jnp.float32)


def _edge_body(g_ref, ef_ref, wef_ref, b1_ref, w2_ref, b2_ref, w3_ref, b3_ref,
               lng_ref, lnb_ref, e_ref, efo_ref):
    ef = ef_ref[...]
    t = g_ref[...] + jnp.dot(ef, wef_ref[...],
                             preferred_element_type=jnp.float32) + b1_ref[...]
    t = jnp.maximum(t, 0.0)
    t = jnp.dot(t, w2_ref[...], preferred_element_type=jnp.float32) + b2_ref[...]
    t = jnp.maximum(t, 0.0)
    e = jnp.dot(t, w3_ref[...], preferred_element_type=jnp.float32) + b3_ref[...]
    m = jnp.mean(e, axis=-1, keepdims=True)
    var = jnp.mean((e - m) ** 2, axis=-1, keepdims=True)
    e = (e - m) * lax.rsqrt(var + 1e-5) * lng_ref[...] + lnb_ref[...]
    e_ref[...] = e
    efo_ref[...] = e + ef


def _node_body(x_ref, a0_ref, a1_ref, wx_ref, wa_ref, b1_ref, w2_ref, b2_ref,
               w3_ref, b3_ref, lng_ref, lnb_ref, o_ref):
    x = x_ref[...]
    agg = a0_ref[...] + a1_ref[...]
    t = (jnp.dot(x, wx_ref[...], preferred_element_type=jnp.float32)
         + jnp.dot(agg, wa_ref[...], preferred_element_type=jnp.float32)
         + b1_ref[...])
    t = jnp.maximum(t, 0.0)
    t = jnp.dot(t, w2_ref[...], preferred_element_type=jnp.float32) + b2_ref[...]
    t = jnp.maximum(t, 0.0)
    nx = jnp.dot(t, w3_ref[...], preferred_element_type=jnp.float32) + b3_ref[...]
    m = jnp.mean(nx, axis=-1, keepdims=True)
    var = jnp.mean((nx - m) ** 2, axis=-1, keepdims=True)
    nx = (nx - m) * lax.rsqrt(var + 1e-5) * lng_ref[...] + lnb_ref[...]
    o_ref[...] = nx + x


def _full(shape):
    nd = len(shape)
    return pl.BlockSpec(shape, lambda i: (0,) * nd)


def _rows(block):
    return pl.BlockSpec((block, D), lambda i: (i, 0))


def _tc_proj(x, wa, wb):
    return pl.pallas_call(
        _proj_body,
        grid=(N // BN,),
        in_specs=[_rows(BN), _full((D, D)), _full((D, D))],
        out_specs=[_rows(BN), _rows(BN)],
        out_shape=[jax.ShapeDtypeStruct((N, D), jnp.float32)] * 2,
    )(x, wa, wb)


def _tc_edge(g, ef, wef, b1, w2, b2, w3, b3, lng, lnb):
    return pl.pallas_call(
        _edge_body,
        grid=(E // BE,),
        in_specs=[_rows(BE), _rows(BE), _full((D, D)), _full((1, D)),
                  _full((D, D)), _full((1, D)), _full((D, D)), _full((1, D)),
                  _full((1, D)), _full((1, D))],
        out_specs=[_rows(BE), _rows(BE)],
        out_shape=[jax.ShapeDtypeStruct((E, D), jnp.float32)] * 2,
    )(g, ef, wef, b1, w2, b2, w3, b3, lng, lnb)


def _tc_node(x, a0, a1, wx, wa, b1, w2, b2, w3, b3, lng, lnb):
    nblk = N // BN
    return pl.pallas_call(
        _node_body,
        grid=(nblk,),
        in_specs=[_rows(BN), _rows(BN), _rows(BN), _full((D, D)), _full((D, D)),
                  _full((1, D)), _full((D, D)), _full((1, D)), _full((D, D)),
                  _full((1, D)), _full((1, D)), _full((1, D))],
        out_specs=_rows(BN),
        out_shape=jax.ShapeDtypeStruct((N, D), jnp.float32),
    )(x, a0, a1, wx, wa, b1, w2, b2, w3, b3, lng, lnb)


def _row2(b):
    return jnp.reshape(b, (1, D))


def kernel(x, edge_index, edge_features, params):
    src = edge_index[0].astype(jnp.int32)
    dst = edge_index[1].astype(jnp.int32)
    ef = edge_features
    for p in params:
        (w1, b1), (w2, b2), (w3, b3) = p["edge_mlp"]
        lng_e, lnb_e = p["edge_ln"]
        (nw1, nb1), (nw2, nb2), (nw3, nb3) = p["node_mlp"]
        lng_n, lnb_n = p["node_ln"]

        u, v = _tc_proj(x, w1[:D], w1[D:2 * D])
        g = _sc_gather_uv(u, v, src, dst)
        e, ef = _tc_edge(g, ef, w1[2 * D:], _row2(b1), w2, _row2(b2),
                         w3, _row2(b3), _row2(lng_e), _row2(lnb_e))
        parts = _sc_scatter_add(e, dst)
        x = _tc_node(x, parts[:N], parts[NP:NP + N], nw1[:D], nw1[D:],
                     _row2(nb1), nw2, _row2(nb2),
                     nw3, _row2(nb3), _row2(lng_n), _row2(lnb_n))
    return x


# speedup vs baseline: 4.9381x; 1.6444x over previous
import functools

import jax
import jax.numpy as jnp
from jax import lax
from jax.experimental import pallas as pl
from jax.experimental.pallas import tpu as pltpu
from jax.experimental.pallas import tpu_sc as plsc

N = 10000
E = 320000
D = 128

NC = 2
NS = 16
NW = NC * NS
EPW = E // NW
K = 80
NCHUNK = EPW // K
NP = 10240
RPS = NP // NS
ZR = 128

BN = 2000
BE = 2000

_MESH = plsc.VectorSubcoreMesh(core_axis_name="c", subcore_axis_name="s")


@functools.partial(
    pl.kernel,
    out_type=jax.ShapeDtypeStruct((E, D), jnp.float32),
    mesh=_MESH,
    scratch_types=[
        pltpu.VMEM((EPW,), jnp.int32),
        pltpu.VMEM((EPW,), jnp.int32),
        pltpu.VMEM((2, K, D), jnp.float32),
        pltpu.VMEM((2, K, D), jnp.float32),
        pltpu.VMEM((2, K, D), jnp.float32),
        pltpu.SemaphoreType.DMA,
        pltpu.SemaphoreType.DMA,
        pltpu.SemaphoreType.DMA,
        pltpu.SemaphoreType.DMA,
        pltpu.SemaphoreType.DMA,
        pltpu.SemaphoreType.DMA,
    ],
)
def _sc_gather_uv(u_hbm, v_hbm, src_hbm, dst_hbm, out_hbm,
                  si, di, ub, vb, ob, su0, su1, sv0, sv1, so0, so1):
    c = lax.axis_index("c")
    s = lax.axis_index("s")
    wid = s * NC + c
    base = wid * EPW
    su = (su0, su1)
    sv = (sv0, sv1)
    so = (so0, so1)

    pltpu.sync_copy(src_hbm.at[pl.ds(base, EPW)], si)
    pltpu.sync_copy(dst_hbm.at[pl.ds(base, EPW)], di)

    def _issue(ch, b):
        pltpu.async_copy(u_hbm.at[si.at[pl.ds(ch * K, K)]], ub.at[b], su[b])
        pltpu.async_copy(v_hbm.at[di.at[pl.ds(ch * K, K)]], vb.at[b], sv[b])

    def _wait_gather(b):
        pltpu.make_async_copy(u_hbm.at[si.at[pl.ds(0, K)]], ub.at[b], su[b]).wait()
        pltpu.make_async_copy(v_hbm.at[di.at[pl.ds(0, K)]], vb.at[b], sv[b]).wait()

    def _wait_out(b):
        pltpu.make_async_copy(ob.at[b], out_hbm.at[pl.ds(base, K), :], so[b]).wait()

    def _add_rows(b):
        @pl.loop(0, K)
        def _row(i):
            @pl.loop(0, D // 16, unroll=8)
            def _col(j):
                sl = pl.ds(j * 16, 16)
                ob[b, i, sl] = ub[b, i, sl] + vb[b, i, sl]

    def _step(ch, b):
        _wait_gather(b)

        @pl.when(ch >= 2)
        def _():
            _wait_out(b)

        _add_rows(b)
        pltpu.async_copy(ob.at[b], out_hbm.at[pl.ds(base + ch * K, K), :], so[b])

        @pl.when(ch + 2 < NCHUNK)
        def _():
            _issue(ch + 2, b)

    _issue(0, 0)
    _issue(1, 1)

    @pl.loop(0, NCHUNK - 1, step=2)
    def _pair(ci):
        _step(ci, 0)
        _step(ci + 1, 1)

    _step(NCHUNK - 1, 0)
    _wait_out(1)
    _wait_out(0)


@functools.partial(
    pl.kernel,
    out_type=jax.ShapeDtypeStruct((2 * NP, D), jnp.float32),
    mesh=_MESH,
    scratch_types=[
        pltpu.VMEM((K,), jnp.int32),
        pltpu.VMEM((K,), jnp.int32),
        pltpu.VMEM((2, K, D), jnp.float32),
        pltpu.VMEM((ZR, D), jnp.float32),
        pltpu.VMEM_SHARED((NP, D), jnp.float32),
        pltpu.SemaphoreType.DMA,
        pltpu.SemaphoreType.DMA,
        pltpu.SemaphoreType.DMA,
        pltpu.SemaphoreType.DMA,
    ],
)
def _sc_scatter_add(e_hbm, dst_hbm, out_hbm, ix0, ix1, rows, zb_v, acc_sh,
                    sr0, sr1, sx0, sx1):
    c = lax.axis_index("c")
    s = lax.axis_index("s")
    wid = s * NC + c
    base = wid * EPW
    ix = (ix0, ix1)
    sr = (sr0, sr1)
    sx = (sx0, sx1)

    @pl.loop(0, ZR)
    def _zrow(i):
        @pl.loop(0, D // 16, unroll=8)
        def _zcol(j):
            zb_v[i, pl.ds(j * 16, 16)] = jnp.zeros((16,), jnp.float32)

    @pl.loop(0, RPS // ZR)
    def _zcopy(k):
        pltpu.sync_copy(zb_v, acc_sh.at[pl.ds(s * RPS + k * ZR, ZR), :])

    plsc.subcore_barrier()

    def _issue(ch, b):
        off = base + ch * K
        pltpu.async_copy(e_hbm.at[pl.ds(off, K), :], rows.at[b], sr[b])
        pltpu.async_copy(dst_hbm.at[pl.ds(off, K)], ix[b], sx[b])

    def _step(ch, b):
        pltpu.make_async_copy(e_hbm.at[pl.ds(base, K), :], rows.at[b], sr[b]).wait()
        pltpu.make_async_copy(dst_hbm.at[pl.ds(base, K)], ix[b], sx[b]).wait()

        pltpu.sync_copy(rows.at[b], acc_sh.at[ix[b]], add=True)

        @pl.when(ch + 2 < NCHUNK)
        def _():
            _issue(ch + 2, b)

    _issue(0, 0)
    _issue(1, 1)

    @pl.loop(0, NCHUNK - 1, step=2)
    def _pair(ci):
        _step(ci, 0)
        _step(ci + 1, 1)

    _step(NCHUNK - 1, 0)

    plsc.subcore_barrier()
    pltpu.sync_copy(acc_sh.at[pl.ds(s * RPS, RPS), :],
                    out_hbm.at[pl.ds(c * NP + s * RPS, RPS), :])


def _proj_body(x_ref, wa_ref, wb_ref, u_ref, v_ref):
    x = x_ref[...]
    u_ref[...] = jnp.dot(x, wa_ref[...], preferred_element_type=jnp.float32)
    v_ref[...] = jnp.dot(x, wb_ref[...], preferred_element_type=jnp.float32)


def _edge_body(g_ref, ef_ref, wef_ref, b1_ref, w2_ref, b2_ref, w3_ref, b3_ref,
               lng_ref, lnb_ref, e_ref, efo_ref):
    ef = ef_ref[...]
    t = g_ref[...] + jnp.dot(ef, wef_ref[...],
                             preferred_element_type=jnp.float32) + b1_ref[...]
    t = jnp.maximum(t, 0.0)
    t = jnp.dot(t, w2_ref[...], preferred_element_type=jnp.float32) + b2_ref[...]
    t = jnp.maximum(t, 0.0)
    e = jnp.dot(t, w3_ref[...], preferred_element_type=jnp.float32) + b3_ref[...]
    m = jnp.mean(e, axis=-1, keepdims=True)
    var = jnp.mean((e - m) ** 2, axis=-1, keepdims=True)
    e = (e - m) * lax.rsqrt(var + 1e-5) * lng_ref[...] + lnb_ref[...]
    e_ref[...] = e
    efo_ref[...] = e + ef


def _node_body(x_ref, a0_ref, a1_ref, wx_ref, wa_ref, b1_ref, w2_ref, b2_ref,
               w3_ref, b3_ref, lng_ref, lnb_ref, o_ref):
    x = x_ref[...]
    agg = a0_ref[...] + a1_ref[...]
    t = (jnp.dot(x, wx_ref[...], preferred_element_type=jnp.float32)
         + jnp.dot(agg, wa_ref[...], preferred_element_type=jnp.float32)
         + b1_ref[...])
    t = jnp.maximum(t, 0.0)
    t = jnp.dot(t, w2_ref[...], preferred_element_type=jnp.float32) + b2_ref[...]
    t = jnp.maximum(t, 0.0)
    nx = jnp.dot(t, w3_ref[...], preferred_element_type=jnp.float32) + b3_ref[...]
    m = jnp.mean(nx, axis=-1, keepdims=True)
    var = jnp.mean((nx - m) ** 2, axis=-1, keepdims=True)
    nx = (nx - m) * lax.rsqrt(var + 1e-5) * lng_ref[...] + lnb_ref[...]
    o_ref[...] = nx + x


def _full(shape):
    nd = len(shape)
    return pl.BlockSpec(shape, lambda i: (0,) * nd)


def _rows(block):
    return pl.BlockSpec((block, D), lambda i: (i, 0))


def _tc_proj(x, wa, wb):
    return pl.pallas_call(
        _proj_body,
        grid=(N // BN,),
        in_specs=[_rows(BN), _full((D, D)), _full((D, D))],
        out_specs=[_rows(BN), _rows(BN)],
        out_shape=[jax.ShapeDtypeStruct((N, D), jnp.float32)] * 2,
    )(x, wa, wb)


def _tc_edge(g, ef, wef, b1, w2, b2, w3, b3, lng, lnb):
    return pl.pallas_call(
        _edge_body,
        grid=(E // BE,),
        in_specs=[_rows(BE), _rows(BE), _full((D, D)), _full((1, D)),
                  _full((D, D)), _full((1, D)), _full((D, D)), _full((1, D)),
                  _full((1, D)), _full((1, D))],
        out_specs=[_rows(BE), _rows(BE)],
        out_shape=[jax.ShapeDtypeStruct((E, D), jnp.float32)] * 2,
    )(g, ef, wef, b1, w2, b2, w3, b3, lng, lnb)


def _tc_node(x, a0, a1, wx, wa, b1, w2, b2, w3, b3, lng, lnb):
    nblk = N // BN
    return pl.pallas_call(
        _node_body,
        grid=(nblk,),
        in_specs=[_rows(BN), _rows(BN), _rows(BN), _full((D, D)), _full((D, D)),
                  _full((1, D)), _full((D, D)), _full((1, D)), _full((D, D)),
                  _full((1, D)), _full((1, D)), _full((1, D))],
        out_specs=_rows(BN),
        out_shape=jax.ShapeDtypeStruct((N, D), jnp.float32),
    )(x, a0, a1, wx, wa, b1, w2, b2, w3, b3, lng, lnb)


def _row2(b):
    return jnp.reshape(b, (1, D))


def kernel(x, edge_index, edge_features, params):
    src = edge_index[0].astype(jnp.int32)
    dst = edge_index[1].astype(jnp.int32)
    ef = edge_features
    for p in params:
        (w1, b1), (w2, b2), (w3, b3) = p["edge_mlp"]
        lng_e, lnb_e = p["edge_ln"]
        (nw1, nb1), (nw2, nb2), (nw3, nb3) = p["node_mlp"]
        lng_n, lnb_n = p["node_ln"]

        u, v = _tc_proj(x, w1[:D], w1[D:2 * D])
        g = _sc_gather_uv(u, v, src, dst)
        e, ef = _tc_edge(g, ef, w1[2 * D:], _row2(b1), w2, _row2(b2),
                         w3, _row2(b3), _row2(lng_e), _row2(lnb_e))
        parts = _sc_scatter_add(e, dst)
        x = _tc_node(x, parts[:N], parts[NP:NP + N], nw1[:D], nw1[D:],
                     _row2(nb1), nw2, _row2(nb2),
                     nw3, _row2(nb3), _row2(lng_n), _row2(lnb_n))
    return x


# speedup vs baseline: 5.4233x; 1.0982x over previous
import functools

import jax
import jax.numpy as jnp
from jax import lax
from jax.experimental import pallas as pl
from jax.experimental.pallas import tpu as pltpu
from jax.experimental.pallas import tpu_sc as plsc

N = 10000
E = 320000
D = 128
E2 = E // 2

NC = 2
NS = 16
NW = NC * NS
NP = 10240
RPS = NP // NS
ZR = 128

BN = 2000
BE = 2000

_MESH = plsc.VectorSubcoreMesh(core_axis_name="c", subcore_axis_name="s")


def _make_sc_gather(EH, K):
    EPW = EH // NW
    NCHUNK = EPW // K
    assert NCHUNK % 2 == 1 and K % 8 == 0 and K <= 128 and EPW % 8 == 0

    @functools.partial(
        pl.kernel,
        out_type=jax.ShapeDtypeStruct((EH, D), jnp.float32),
        mesh=_MESH,
        scratch_types=[
            pltpu.VMEM((EPW,), jnp.int32),
            pltpu.VMEM((EPW,), jnp.int32),
            pltpu.VMEM((2, K, D), jnp.float32),
            pltpu.VMEM((2, K, D), jnp.float32),
            pltpu.VMEM((2, K, D), jnp.float32),
            pltpu.SemaphoreType.DMA,
            pltpu.SemaphoreType.DMA,
            pltpu.SemaphoreType.DMA,
            pltpu.SemaphoreType.DMA,
            pltpu.SemaphoreType.DMA,
            pltpu.SemaphoreType.DMA,
        ],
    )
    def _sc_gather_uv(u_hbm, v_hbm, src_hbm, dst_hbm, out_hbm,
                      si, di, ub, vb, ob, su0, su1, sv0, sv1, so0, so1):
        c = lax.axis_index("c")
        s = lax.axis_index("s")
        wid = s * NC + c
        base = wid * EPW
        su = (su0, su1)
        sv = (sv0, sv1)
        so = (so0, so1)

        pltpu.sync_copy(src_hbm.at[pl.ds(base, EPW)], si)
        pltpu.sync_copy(dst_hbm.at[pl.ds(base, EPW)], di)

        def _issue(ch, b):
            pltpu.async_copy(u_hbm.at[si.at[pl.ds(ch * K, K)]], ub.at[b], su[b])
            pltpu.async_copy(v_hbm.at[di.at[pl.ds(ch * K, K)]], vb.at[b], sv[b])

        def _wait_gather(b):
            pltpu.make_async_copy(u_hbm.at[si.at[pl.ds(0, K)]], ub.at[b],
                                  su[b]).wait()
            pltpu.make_async_copy(v_hbm.at[di.at[pl.ds(0, K)]], vb.at[b],
                                  sv[b]).wait()

        def _wait_out(b):
            pltpu.make_async_copy(ob.at[b], out_hbm.at[pl.ds(base, K), :],
                                  so[b]).wait()

        def _add_rows(b):
            @pl.loop(0, K)
            def _row(i):
                @pl.loop(0, D // 16, unroll=8)
                def _col(j):
                    sl = pl.ds(j * 16, 16)
                    ob[b, i, sl] = ub[b, i, sl] + vb[b, i, sl]

        def _step(ch, b):
            _wait_gather(b)

            @pl.when(ch >= 2)
            def _():
                _wait_out(b)

            _add_rows(b)
            pltpu.async_copy(ob.at[b], out_hbm.at[pl.ds(base + ch * K, K), :],
                             so[b])

            @pl.when(ch + 2 < NCHUNK)
            def _():
                _issue(ch + 2, b)

        _issue(0, 0)
        _issue(1, 1)

        @pl.loop(0, NCHUNK - 1, step=2)
        def _pair(ci):
            _step(ci, 0)
            _step(ci + 1, 1)

        _step(NCHUNK - 1, 0)
        _wait_out(1)
        _wait_out(0)

    return _sc_gather_uv


def _make_sc_scatter(EH, K):
    EPW = EH // NW
    NCHUNK = EPW // K
    assert NCHUNK % 2 == 1 and K % 8 == 0 and K <= 128 and EPW % 8 == 0

    @functools.partial(
        pl.kernel,
        out_type=jax.ShapeDtypeStruct((2 * NP, D), jnp.float32),
        mesh=_MESH,
        scratch_types=[
            pltpu.VMEM((K,), jnp.int32),
            pltpu.VMEM((K,), jnp.int32),
            pltpu.VMEM((2, K, D), jnp.float32),
            pltpu.VMEM((ZR, D), jnp.float32),
            pltpu.VMEM_SHARED((NP, D), jnp.float32),
            pltpu.SemaphoreType.DMA,
            pltpu.SemaphoreType.DMA,
            pltpu.SemaphoreType.DMA,
            pltpu.SemaphoreType.DMA,
        ],
    )
    def _sc_scatter_add(e_hbm, dst_hbm, out_hbm, ix0, ix1, rows, zb_v, acc_sh,
                        sr0, sr1, sx0, sx1):
        c = lax.axis_index("c")
        s = lax.axis_index("s")
        wid = s * NC + c
        base = wid * EPW
        ix = (ix0, ix1)
        sr = (sr0, sr1)
        sx = (sx0, sx1)

        @pl.loop(0, ZR)
        def _zrow(i):
            @pl.loop(0, D // 16, unroll=8)
            def _zcol(j):
                zb_v[i, pl.ds(j * 16, 16)] = jnp.zeros((16,), jnp.float32)

        @pl.loop(0, RPS // ZR)
        def _zcopy(k):
            pltpu.sync_copy(zb_v, acc_sh.at[pl.ds(s * RPS + k * ZR, ZR), :])

        plsc.subcore_barrier()

        def _issue(ch, b):
            off = base + ch * K
            pltpu.async_copy(e_hbm.at[pl.ds(off, K), :], rows.at[b], sr[b])
            pltpu.async_copy(dst_hbm.at[pl.ds(off, K)], ix[b], sx[b])

        def _step(ch, b):
            pltpu.make_async_copy(e_hbm.at[pl.ds(base, K), :], rows.at[b],
                                  sr[b]).wait()
            pltpu.make_async_copy(dst_hbm.at[pl.ds(base, K)], ix[b],
                                  sx[b]).wait()

            pltpu.sync_copy(rows.at[b], acc_sh.at[ix[b]], add=True)

            @pl.when(ch + 2 < NCHUNK)
            def _():
                _issue(ch + 2, b)

        _issue(0, 0)
        _issue(1, 1)

        @pl.loop(0, NCHUNK - 1, step=2)
        def _pair(ci):
            _step(ci, 0)
            _step(ci + 1, 1)

        _step(NCHUNK - 1, 0)

        plsc.subcore_barrier()
        pltpu.sync_copy(acc_sh.at[pl.ds(s * RPS, RPS), :],
                        out_hbm.at[pl.ds(c * NP + s * RPS, RPS), :])

    return _sc_scatter_add


_sc_gather_half = _make_sc_gather(E2, 40)
_sc_scatter_half = _make_sc_scatter(E2, 40)


def _proj_body(x_ref, wa_ref, wb_ref, u_ref, v_ref):
    x = x_ref[...]
    u_ref[...] = jnp.dot(x, wa_ref[...], preferred_element_type=jnp.float32)
    v_ref[...] = jnp.dot(x, wb_ref[...], preferred_element_type=jnp.float32)


def _edge_core(g_ref, ef_ref, wef_ref, b1_ref, w2_ref, b2_ref, w3_ref, b3_ref,
               lng_ref, lnb_ref):
    ef = ef_ref[...]
    t = g_ref[...] + jnp.dot(
        ef, wef_ref[...], preferred_element_type=jnp.float32) + b1_ref[...]
    t = jnp.maximum(t, 0.0)
    t = jnp.dot(t, w2_ref[...], preferred_element_type=jnp.float32) + b2_ref[...]
    t = jnp.maximum(t, 0.0)
    e = jnp.dot(t, w3_ref[...], preferred_element_type=jnp.float32) + b3_ref[...]
    m = jnp.mean(e, axis=-1, keepdims=True)
    var = jnp.mean((e - m) ** 2, axis=-1, keepdims=True)
    return (e - m) * lax.rsqrt(var + 1e-5) * lng_ref[...] + lnb_ref[...], ef


def _edge_body(g_ref, ef_ref, wef_ref, b1_ref, w2_ref, b2_ref, w3_ref, b3_ref,
               lng_ref, lnb_ref, e_ref, efo_ref):
    e, ef = _edge_core(g_ref, ef_ref, wef_ref, b1_ref, w2_ref, b2_ref, w3_ref,
                       b3_ref, lng_ref, lnb_ref)
    e_ref[...] = e
    efo_ref[...] = e + ef


def _edge_body_last(g_ref, ef_ref, wef_ref, b1_ref, w2_ref, b2_ref, w3_ref,
                    b3_ref, lng_ref, lnb_ref, e_ref):
    e, _ = _edge_core(g_ref, ef_ref, wef_ref, b1_ref, w2_ref, b2_ref, w3_ref,
                      b3_ref, lng_ref, lnb_ref)
    e_ref[...] = e


def _node_body(x_ref, a0_ref, a1_ref, a2_ref, a3_ref, wx_ref, wa_ref, b1_ref,
               w2_ref, b2_ref, w3_ref, b3_ref, lng_ref, lnb_ref, o_ref):
    x = x_ref[...]
    agg = (a0_ref[...] + a1_ref[...]) + (a2_ref[...] + a3_ref[...])
    t = (jnp.dot(x, wx_ref[...], preferred_element_type=jnp.float32)
         + jnp.dot(agg, wa_ref[...], preferred_element_type=jnp.float32)
         + b1_ref[...])
    t = jnp.maximum(t, 0.0)
    t = jnp.dot(t, w2_ref[...], preferred_element_type=jnp.float32) + b2_ref[...]
    t = jnp.maximum(t, 0.0)
    nx = jnp.dot(t, w3_ref[...], preferred_element_type=jnp.float32) + b3_ref[...]
    m = jnp.mean(nx, axis=-1, keepdims=True)
    var = jnp.mean((nx - m) ** 2, axis=-1, keepdims=True)
    nx = (nx - m) * lax.rsqrt(var + 1e-5) * lng_ref[...] + lnb_ref[...]
    o_ref[...] = nx + x


def _full(shape):
    nd = len(shape)
    return pl.BlockSpec(shape, lambda i: (0,) * nd)


def _rows(block, off=0):
    return pl.BlockSpec((block, D), lambda i, off=off: (i + off, 0))


def _tc_proj(x, wa, wb):
    return pl.pallas_call(
        _proj_body,
        grid=(N // BN,),
        in_specs=[_rows(BN), _full((D, D)), _full((D, D))],
        out_specs=[_rows(BN), _rows(BN)],
        out_shape=[jax.ShapeDtypeStruct((N, D), jnp.float32)] * 2,
    )(x, wa, wb)


def _tc_edge(g, ef, wef, b1, w2, b2, w3, b3, lng, lnb, ef_blk_off, last):
    in_specs = [_rows(BE), _rows(BE, ef_blk_off), _full((D, D)), _full((1, D)),
                _full((D, D)), _full((1, D)), _full((D, D)), _full((1, D)),
                _full((1, D)), _full((1, D))]
    if last:
        return pl.pallas_call(
            _edge_body_last,
            grid=(E2 // BE,),
            in_specs=in_specs,
            out_specs=_rows(BE),
            out_shape=jax.ShapeDtypeStruct((E2, D), jnp.float32),
        )(g, ef, wef, b1, w2, b2, w3, b3, lng, lnb), None
    e, efo = pl.pallas_call(
        _edge_body,
        grid=(E2 // BE,),
        in_specs=in_specs,
        out_specs=[_rows(BE), _rows(BE)],
        out_shape=[jax.ShapeDtypeStruct((E2, D), jnp.float32)] * 2,
    )(g, ef, wef, b1, w2, b2, w3, b3, lng, lnb)
    return e, efo


def _tc_node(x, pa, pb, wx, wa, b1, w2, b2, w3, b3, lng, lnb):
    return pl.pallas_call(
        _node_body,
        grid=(N // BN,),
        in_specs=[_rows(BN), _rows(BN), _rows(BN), _rows(BN), _rows(BN),
                  _full((D, D)), _full((D, D)), _full((1, D)), _full((D, D)),
                  _full((1, D)), _full((D, D)), _full((1, D)), _full((1, D)),
                  _full((1, D))],
        out_specs=_rows(BN),
        out_shape=jax.ShapeDtypeStruct((N, D), jnp.float32),
    )(x, pa[:N], pa[NP:NP + N], pb[:N], pb[NP:NP + N],
      wx, wa, b1, w2, b2, w3, b3, lng, lnb)


def _row2(b):
    return jnp.reshape(b, (1, D))


def kernel(x, edge_index, edge_features, params):
    src = edge_index[0].astype(jnp.int32)
    dst = edge_index[1].astype(jnp.int32)
    src_h = (src[:E2], src[E2:])
    dst_h = (dst[:E2], dst[E2:])
    ef_h = (edge_features, edge_features)
    ef_off = (0, E2 // BE)
    nsteps = len(params)
    for si, p in enumerate(params):
        (w1, b1), (w2, b2), (w3, b3) = p["edge_mlp"]
        lng_e, lnb_e = p["edge_ln"]
        (nw1, nb1), (nw2, nb2), (nw3, nb3) = p["node_mlp"]
        lng_n, lnb_n = p["node_ln"]
        last = si == nsteps - 1

        u, v = _tc_proj(x, w1[:D], w1[D:2 * D])
        g0 = _sc_gather_half(u, v, src_h[0], dst_h[0])
        g1 = _sc_gather_half(u, v, src_h[1], dst_h[1])
        e0, efo0 = _tc_edge(g0, ef_h[0], w1[2 * D:], _row2(b1), w2, _row2(b2),
                            w3, _row2(b3), _row2(lng_e), _row2(lnb_e),
                            ef_off[0], last)
        p0 = _sc_scatter_half(e0, dst_h[0])
        e1, efo1 = _tc_edge(g1, ef_h[1], w1[2 * D:], _row2(b1), w2, _row2(b2),
                            w3, _row2(b3), _row2(lng_e), _row2(lnb_e),
                            ef_off[1], last)
        p1 = _sc_scatter_half(e1, dst_h[1])
        x = _tc_node(x, p0, p1, nw1[:D], nw1[D:], _row2(nb1), nw2, _row2(nb2),
                     nw3, _row2(nb3), _row2(lng_n), _row2(lnb_n))
        ef_h = (efo0, efo1)
        ef_off = (0, 0)
    return x


# speedup vs baseline: 5.7183x; 1.0544x over previous
import functools

import jax
import jax.numpy as jnp
from jax import lax
from jax.experimental import pallas as pl
from jax.experimental.pallas import tpu as pltpu
from jax.experimental.pallas import tpu_sc as plsc

N = 10000
E = 320000
D = 128
E2 = E // 2

NC = 2
NS = 16
NW = NC * NS
NP = 10240
RPS = NP // NS
ZR = 64

BN = 2000
BE = 2000

_MESH = plsc.VectorSubcoreMesh(core_axis_name="c", subcore_axis_name="s")


def _make_sc_gather(EH, K):
    EPW = EH // NW
    NCHUNK = EPW // K
    TAIL = EPW - NCHUNK * K
    assert NCHUNK % 2 == 1 and K % 8 == 0 and K <= 128 and EPW % 8 == 0
    assert TAIL % 8 == 0 and (NCHUNK * K) % 8 == 0

    @functools.partial(
        pl.kernel,
        out_type=jax.ShapeDtypeStruct((EH, D), jnp.float32),
        mesh=_MESH,
        scratch_types=[
            pltpu.VMEM((EPW,), jnp.int32),
            pltpu.VMEM((EPW,), jnp.int32),
            pltpu.VMEM((2, K, D), jnp.float32),
            pltpu.VMEM((2, K, D), jnp.float32),
            pltpu.VMEM((2, K, D), jnp.float32),
            pltpu.SemaphoreType.DMA,
            pltpu.SemaphoreType.DMA,
            pltpu.SemaphoreType.DMA,
            pltpu.SemaphoreType.DMA,
            pltpu.SemaphoreType.DMA,
            pltpu.SemaphoreType.DMA,
        ],
    )
    def _sc_gather_uv(u_hbm, v_hbm, src_hbm, dst_hbm, out_hbm,
                      si, di, ub, vb, ob, su0, su1, sv0, sv1, so0, so1):
        c = lax.axis_index("c")
        s = lax.axis_index("s")
        wid = s * NC + c
        base = wid * EPW
        su = (su0, su1)
        sv = (sv0, sv1)
        so = (so0, so1)

        pltpu.sync_copy(src_hbm.at[pl.ds(base, EPW)], si)
        pltpu.sync_copy(dst_hbm.at[pl.ds(base, EPW)], di)

        def _issue(ch, b):
            pltpu.async_copy(u_hbm.at[si.at[pl.ds(ch * K, K)]], ub.at[b], su[b])
            pltpu.async_copy(v_hbm.at[di.at[pl.ds(ch * K, K)]], vb.at[b], sv[b])

        def _wait_gather(b):
            pltpu.make_async_copy(u_hbm.at[si.at[pl.ds(0, K)]], ub.at[b],
                                  su[b]).wait()
            pltpu.make_async_copy(v_hbm.at[di.at[pl.ds(0, K)]], vb.at[b],
                                  sv[b]).wait()

        def _wait_out(b):
            pltpu.make_async_copy(ob.at[b], out_hbm.at[pl.ds(base, K), :],
                                  so[b]).wait()

        def _add_rows(b, nrows):
            @pl.loop(0, nrows)
            def _row(i):
                @pl.loop(0, D // 16, unroll=8)
                def _col(j):
                    sl = pl.ds(j * 16, 16)
                    ob[b, i, sl] = ub[b, i, sl] + vb[b, i, sl]

        def _step(ch, b):
            _wait_gather(b)

            @pl.when(ch >= 2)
            def _():
                _wait_out(b)

            _add_rows(b, K)
            pltpu.async_copy(ob.at[b], out_hbm.at[pl.ds(base + ch * K, K), :],
                             so[b])

            @pl.when(ch + 2 < NCHUNK)
            def _():
                _issue(ch + 2, b)

        _issue(0, 0)
        _issue(1, 1)

        @pl.loop(0, NCHUNK - 1, step=2)
        def _pair(ci):
            _step(ci, 0)
            _step(ci + 1, 1)

        _step(NCHUNK - 1, 0)
        _wait_out(1)
        _wait_out(0)

        if TAIL:
            toff = NCHUNK * K
            pltpu.async_copy(u_hbm.at[si.at[pl.ds(toff, TAIL)]],
                             ub.at[0, pl.ds(0, TAIL), :], su[0])
            pltpu.async_copy(v_hbm.at[di.at[pl.ds(toff, TAIL)]],
                             vb.at[0, pl.ds(0, TAIL), :], sv[0])
            pltpu.make_async_copy(u_hbm.at[si.at[pl.ds(0, TAIL)]],
                                  ub.at[0, pl.ds(0, TAIL), :], su[0]).wait()
            pltpu.make_async_copy(v_hbm.at[di.at[pl.ds(0, TAIL)]],
                                  vb.at[0, pl.ds(0, TAIL), :], sv[0]).wait()
            _add_rows(0, TAIL)
            pltpu.sync_copy(ob.at[0, pl.ds(0, TAIL), :],
                            out_hbm.at[pl.ds(base + toff, TAIL), :])

    return _sc_gather_uv


def _make_sc_scatter(EH, K):
    EPW = EH // NW
    NCHUNK = EPW // K
    TAIL = EPW - NCHUNK * K
    assert NCHUNK % 2 == 1 and K % 8 == 0 and K <= 128 and EPW % 8 == 0
    assert TAIL % 8 == 0 and (NCHUNK * K) % 8 == 0

    @functools.partial(
        pl.kernel,
        out_type=jax.ShapeDtypeStruct((2 * NP, D), jnp.float32),
        mesh=_MESH,
        scratch_types=[
            pltpu.VMEM((K,), jnp.int32),
            pltpu.VMEM((K,), jnp.int32),
            pltpu.VMEM((max(TAIL, 8),), jnp.int32),
            pltpu.VMEM((2, K, D), jnp.float32),
            pltpu.VMEM((ZR, D), jnp.float32),
            pltpu.VMEM_SHARED((NP, D), jnp.float32),
            pltpu.SemaphoreType.DMA,
            pltpu.SemaphoreType.DMA,
            pltpu.SemaphoreType.DMA,
            pltpu.SemaphoreType.DMA,
        ],
    )
    def _sc_scatter_add(e_hbm, dst_hbm, out_hbm, ix0, ix1, ixt, rows, zb_v,
                        acc_sh, sr0, sr1, sx0, sx1):
        c = lax.axis_index("c")
        s = lax.axis_index("s")
        wid = s * NC + c
        base = wid * EPW
        ix = (ix0, ix1)
        sr = (sr0, sr1)
        sx = (sx0, sx1)

        @pl.loop(0, ZR)
        def _zrow(i):
            @pl.loop(0, D // 16, unroll=8)
            def _zcol(j):
                zb_v[i, pl.ds(j * 16, 16)] = jnp.zeros((16,), jnp.float32)

        @pl.loop(0, RPS // ZR)
        def _zcopy(k):
            pltpu.sync_copy(zb_v, acc_sh.at[pl.ds(s * RPS + k * ZR, ZR), :])

        plsc.subcore_barrier()

        def _issue(ch, b):
            off = base + ch * K
            pltpu.async_copy(e_hbm.at[pl.ds(off, K), :], rows.at[b], sr[b])
            pltpu.async_copy(dst_hbm.at[pl.ds(off, K)], ix[b], sx[b])

        def _step(ch, b):
            pltpu.make_async_copy(e_hbm.at[pl.ds(base, K), :], rows.at[b],
                                  sr[b]).wait()
            pltpu.make_async_copy(dst_hbm.at[pl.ds(base, K)], ix[b],
                                  sx[b]).wait()

            pltpu.sync_copy(rows.at[b], acc_sh.at[ix[b]], add=True)

            @pl.when(ch + 2 < NCHUNK)
            def _():
                _issue(ch + 2, b)

        _issue(0, 0)
        _issue(1, 1)

        @pl.loop(0, NCHUNK - 1, step=2)
        def _pair(ci):
            _step(ci, 0)
            _step(ci + 1, 1)

        _step(NCHUNK - 1, 0)

        if TAIL:
            toff = base + NCHUNK * K
            pltpu.sync_copy(dst_hbm.at[pl.ds(toff, TAIL)], ixt)
            pltpu.sync_copy(e_hbm.at[pl.ds(toff, TAIL), :],
                            rows.at[0, pl.ds(0, TAIL), :])
            pltpu.sync_copy(rows.at[0, pl.ds(0, TAIL), :], acc_sh.at[ixt],
                            add=True)

        plsc.subcore_barrier()
        pltpu.sync_copy(acc_sh.at[pl.ds(s * RPS, RPS), :],
                        out_hbm.at[pl.ds(c * NP + s * RPS, RPS), :])

    return _sc_scatter_add


_sc_gather_half = _make_sc_gather(E2, 128)
_sc_scatter_half = _make_sc_scatter(E2, 128)


def _proj_body(x_ref, wa_ref, wb_ref, u_ref, v_ref):
    x = x_ref[...]
    u_ref[...] = jnp.dot(x, wa_ref[...], preferred_element_type=jnp.float32)
    v_ref[...] = jnp.dot(x, wb_ref[...], preferred_element_type=jnp.float32)


def _edge_core(g_ref, ef_ref, wef_ref, b1_ref, w2_ref, b2_ref, w3_ref, b3_ref,
               lng_ref, lnb_ref):
    ef = ef_ref[...]
    t = g_ref[...] + jnp.dot(
        ef, wef_ref[...], preferred_element_type=jnp.float32) + b1_ref[...]
    t = jnp.maximum(t, 0.0)
    t = jnp.dot(t, w2_ref[...], preferred_element_type=jnp.float32) + b2_ref[...]
    t = jnp.maximum(t, 0.0)
    e = jnp.dot(t, w3_ref[...], preferred_element_type=jnp.float32) + b3_ref[...]
    m = jnp.mean(e, axis=-1, keepdims=True)
    var = jnp.mean((e - m) ** 2, axis=-1, keepdims=True)
    return (e - m) * lax.rsqrt(var + 1e-5) * lng_ref[...] + lnb_ref[...], ef


def _edge_body(g_ref, ef_ref, wef_ref, b1_ref, w2_ref, b2_ref, w3_ref, b3_ref,
               lng_ref, lnb_ref, e_ref, efo_ref):
    e, ef = _edge_core(g_ref, ef_ref, wef_ref, b1_ref, w2_ref, b2_ref, w3_ref,
                       b3_ref, lng_ref, lnb_ref)
    e_ref[...] = e
    efo_ref[...] = e + ef


def _edge_body_last(g_ref, ef_ref, wef_ref, b1_ref, w2_ref, b2_ref, w3_ref,
                    b3_ref, lng_ref, lnb_ref, e_ref):
    e, _ = _edge_core(g_ref, ef_ref, wef_ref, b1_ref, w2_ref, b2_ref, w3_ref,
                      b3_ref, lng_ref, lnb_ref)
    e_ref[...] = e


def _node_body(x_ref, a0_ref, a1_ref, a2_ref, a3_ref, wx_ref, wa_ref, b1_ref,
               w2_ref, b2_ref, w3_ref, b3_ref, lng_ref, lnb_ref, o_ref):
    x = x_ref[...]
    agg = (a0_ref[...] + a1_ref[...]) + (a2_ref[...] + a3_ref[...])
    t = (jnp.dot(x, wx_ref[...], preferred_element_type=jnp.float32)
         + jnp.dot(agg, wa_ref[...], preferred_element_type=jnp.float32)
         + b1_ref[...])
    t = jnp.maximum(t, 0.0)
    t = jnp.dot(t, w2_ref[...], preferred_element_type=jnp.float32) + b2_ref[...]
    t = jnp.maximum(t, 0.0)
    nx = jnp.dot(t, w3_ref[...], preferred_element_type=jnp.float32) + b3_ref[...]
    m = jnp.mean(nx, axis=-1, keepdims=True)
    var = jnp.mean((nx - m) ** 2, axis=-1, keepdims=True)
    nx = (nx - m) * lax.rsqrt(var + 1e-5) * lng_ref[...] + lnb_ref[...]
    o_ref[...] = nx + x


def _full(shape):
    nd = len(shape)
    return pl.BlockSpec(shape, lambda i: (0,) * nd)


def _rows(block, off=0):
    return pl.BlockSpec((block, D), lambda i, off=off: (i + off, 0))


def _tc_proj(x, wa, wb):
    return pl.pallas_call(
        _proj_body,
        grid=(N // BN,),
        in_specs=[_rows(BN), _full((D, D)), _full((D, D))],
        out_specs=[_rows(BN), _rows(BN)],
        out_shape=[jax.ShapeDtypeStruct((N, D), jnp.float32)] * 2,
    )(x, wa, wb)


def _tc_edge(g, ef, wef, b1, w2, b2, w3, b3, lng, lnb, ef_blk_off, last):
    in_specs = [_rows(BE), _rows(BE, ef_blk_off), _full((D, D)), _full((1, D)),
                _full((D, D)), _full((1, D)), _full((D, D)), _full((1, D)),
                _full((1, D)), _full((1, D))]
    if last:
        return pl.pallas_call(
            _edge_body_last,
            grid=(E2 // BE,),
            in_specs=in_specs,
            out_specs=_rows(BE),
            out_shape=jax.ShapeDtypeStruct((E2, D), jnp.float32),
        )(g, ef, wef, b1, w2, b2, w3, b3, lng, lnb), None
    e, efo = pl.pallas_call(
        _edge_body,
        grid=(E2 // BE,),
        in_specs=in_specs,
        out_specs=[_rows(BE), _rows(BE)],
        out_shape=[jax.ShapeDtypeStruct((E2, D), jnp.float32)] * 2,
    )(g, ef, wef, b1, w2, b2, w3, b3, lng, lnb)
    return e, efo


def _tc_node(x, pa, pb, wx, wa, b1, w2, b2, w3, b3, lng, lnb):
    return pl.pallas_call(
        _node_body,
        grid=(N // BN,),
        in_specs=[_rows(BN), _rows(BN), _rows(BN), _rows(BN), _rows(BN),
                  _full((D, D)), _full((D, D)), _full((1, D)), _full((D, D)),
                  _full((1, D)), _full((D, D)), _full((1, D)), _full((1, D)),
                  _full((1, D))],
        out_specs=_rows(BN),
        out_shape=jax.ShapeDtypeStruct((N, D), jnp.float32),
    )(x, pa[:N], pa[NP:NP + N], pb[:N], pb[NP:NP + N],
      wx, wa, b1, w2, b2, w3, b3, lng, lnb)


def _row2(b):
    return jnp.reshape(b, (1, D))


def kernel(x, edge_index, edge_features, params):
    src = edge_index[0].astype(jnp.int32)
    dst = edge_index[1].astype(jnp.int32)
    src_h = (src[:E2], src[E2:])
    dst_h = (dst[:E2], dst[E2:])
    ef_h = (edge_features, edge_features)
    ef_off = (0, E2 // BE)
    nsteps = len(params)
    for si, p in enumerate(params):
        (w1, b1), (w2, b2), (w3, b3) = p["edge_mlp"]
        lng_e, lnb_e = p["edge_ln"]
        (nw1, nb1), (nw2, nb2), (nw3, nb3) = p["node_mlp"]
        lng_n, lnb_n = p["node_ln"]
        last = si == nsteps - 1

        u, v = _tc_proj(x, w1[:D], w1[D:2 * D])
        g0 = _sc_gather_half(u, v, src_h[0], dst_h[0])
        g1 = _sc_gather_half(u, v, src_h[1], dst_h[1])
        e0, efo0 = _tc_edge(g0, ef_h[0], w1[2 * D:], _row2(b1), w2, _row2(b2),
                            w3, _row2(b3), _row2(lng_e), _row2(lnb_e),
                            ef_off[0], last)
        p0 = _sc_scatter_half(e0, dst_h[0])
        e1, efo1 = _tc_edge(g1, ef_h[1], w1[2 * D:], _row2(b1), w2, _row2(b2),
                            w3, _row2(b3), _row2(lng_e), _row2(lnb_e),
                            ef_off[1], last)
        p1 = _sc_scatter_half(e1, dst_h[1])
        x = _tc_node(x, p0, p1, nw1[:D], nw1[D:], _row2(nb1), nw2, _row2(nb2),
                     nw3, _row2(nb3), _row2(lng_n), _row2(lnb_n))
        ef_h = (efo0, efo1)
        ef_off = (0, 0)
    return x


# speedup vs baseline: 5.8446x; 1.0221x over previous
import functools

import jax
import jax.numpy as jnp
from jax import lax
from jax.experimental import pallas as pl
from jax.experimental.pallas import tpu as pltpu
from jax.experimental.pallas import tpu_sc as plsc

N = 10000
E = 320000
D = 128
E2 = E // 2

NC = 2
NS = 16
NW = NC * NS
NP = 10240
RPS = NP // NS
ZR = 64

BN = 2000
BE = 2000

_MESH = plsc.VectorSubcoreMesh(core_axis_name="c", subcore_axis_name="s")


def _make_sc_gather(EH, K):
    EPW = EH // NW
    NCHUNK = EPW // K
    TAIL = EPW - NCHUNK * K
    assert NCHUNK % 2 == 1 and K % 8 == 0 and K <= 128 and EPW % 8 == 0
    assert TAIL % 8 == 0 and (NCHUNK * K) % 8 == 0

    @functools.partial(
        pl.kernel,
        out_type=jax.ShapeDtypeStruct((EH, D), jnp.float32),
        mesh=_MESH,
        scratch_types=[
            pltpu.VMEM((EPW,), jnp.int32),
            pltpu.VMEM((EPW,), jnp.int32),
            pltpu.VMEM((2, K, D), jnp.float32),
            pltpu.VMEM((2, K, D), jnp.float32),
            pltpu.VMEM((2, K, D), jnp.float32),
            pltpu.SemaphoreType.DMA,
            pltpu.SemaphoreType.DMA,
            pltpu.SemaphoreType.DMA,
            pltpu.SemaphoreType.DMA,
            pltpu.SemaphoreType.DMA,
            pltpu.SemaphoreType.DMA,
        ],
    )
    def _sc_gather_uv(u_hbm, v_hbm, src_hbm, dst_hbm, out_hbm,
                      si, di, ub, vb, ob, su0, su1, sv0, sv1, so0, so1):
        c = lax.axis_index("c")
        s = lax.axis_index("s")
        wid = s * NC + c
        base = wid * EPW
        su = (su0, su1)
        sv = (sv0, sv1)
        so = (so0, so1)

        pltpu.sync_copy(src_hbm.at[pl.ds(base, EPW)], si)
        pltpu.sync_copy(dst_hbm.at[pl.ds(base, EPW)], di)

        def _issue(ch, b):
            pltpu.async_copy(u_hbm.at[si.at[pl.ds(ch * K, K)]], ub.at[b], su[b])
            pltpu.async_copy(v_hbm.at[di.at[pl.ds(ch * K, K)]], vb.at[b], sv[b])

        def _wait_gather(b):
            pltpu.make_async_copy(u_hbm.at[si.at[pl.ds(0, K)]], ub.at[b],
                                  su[b]).wait()
            pltpu.make_async_copy(v_hbm.at[di.at[pl.ds(0, K)]], vb.at[b],
                                  sv[b]).wait()

        def _wait_out(b):
            pltpu.make_async_copy(ob.at[b], out_hbm.at[pl.ds(base, K), :],
                                  so[b]).wait()

        def _add_rows(b, nrows):
            @pl.loop(0, nrows)
            def _row(i):
                @pl.loop(0, D // 16, unroll=8)
                def _col(j):
                    sl = pl.ds(j * 16, 16)
                    ob[b, i, sl] = ub[b, i, sl] + vb[b, i, sl]

        def _step(ch, b):
            _wait_gather(b)

            @pl.when(ch >= 2)
            def _():
                _wait_out(b)

            _add_rows(b, K)
            pltpu.async_copy(ob.at[b], out_hbm.at[pl.ds(base + ch * K, K), :],
                             so[b])

            @pl.when(ch + 2 < NCHUNK)
            def _():
                _issue(ch + 2, b)

        _issue(0, 0)
        _issue(1, 1)

        @pl.loop(0, NCHUNK - 1, step=2)
        def _pair(ci):
            _step(ci, 0)
            _step(ci + 1, 1)

        _step(NCHUNK - 1, 0)
        _wait_out(1)
        _wait_out(0)

        if TAIL:
            toff = NCHUNK * K
            pltpu.async_copy(u_hbm.at[si.at[pl.ds(toff, TAIL)]],
                             ub.at[0, pl.ds(0, TAIL), :], su[0])
            pltpu.async_copy(v_hbm.at[di.at[pl.ds(toff, TAIL)]],
                             vb.at[0, pl.ds(0, TAIL), :], sv[0])
            pltpu.make_async_copy(u_hbm.at[si.at[pl.ds(0, TAIL)]],
                                  ub.at[0, pl.ds(0, TAIL), :], su[0]).wait()
            pltpu.make_async_copy(v_hbm.at[di.at[pl.ds(0, TAIL)]],
                                  vb.at[0, pl.ds(0, TAIL), :], sv[0]).wait()
            _add_rows(0, TAIL)
            pltpu.sync_copy(ob.at[0, pl.ds(0, TAIL), :],
                            out_hbm.at[pl.ds(base + toff, TAIL), :])

    return _sc_gather_uv


def _make_sc_scatter(EH, K):
    EPW = EH // NW
    NCHUNK = EPW // K
    TAIL = EPW - NCHUNK * K
    assert NCHUNK % 2 == 1 and K % 8 == 0 and K <= 128 and EPW % 8 == 0
    assert TAIL % 8 == 0 and (NCHUNK * K) % 8 == 0

    @functools.partial(
        pl.kernel,
        out_type=jax.ShapeDtypeStruct((2 * NP, D), jnp.float32),
        mesh=_MESH,
        scratch_types=[
            pltpu.VMEM((K,), jnp.int32),
            pltpu.VMEM((K,), jnp.int32),
            pltpu.VMEM((max(TAIL, 8),), jnp.int32),
            pltpu.VMEM((2, K, D), jnp.float32),
            pltpu.VMEM((ZR, D), jnp.float32),
            pltpu.VMEM_SHARED((NP, D), jnp.float32),
            pltpu.SemaphoreType.DMA,
            pltpu.SemaphoreType.DMA,
            pltpu.SemaphoreType.DMA,
            pltpu.SemaphoreType.DMA,
        ],
    )
    def _sc_scatter_add(e_hbm, dst_hbm, out_hbm, ix0, ix1, ixt, rows, zb_v,
                        acc_sh, sr0, sr1, sx0, sx1):
        c = lax.axis_index("c")
        s = lax.axis_index("s")
        wid = s * NC + c
        base = wid * EPW
        ix = (ix0, ix1)
        sr = (sr0, sr1)
        sx = (sx0, sx1)

        @pl.loop(0, ZR)
        def _zrow(i):
            @pl.loop(0, D // 16, unroll=8)
            def _zcol(j):
                zb_v[i, pl.ds(j * 16, 16)] = jnp.zeros((16,), jnp.float32)

        @pl.loop(0, RPS // ZR)
        def _zcopy(k):
            pltpu.sync_copy(zb_v, acc_sh.at[pl.ds(s * RPS + k * ZR, ZR), :])

        plsc.subcore_barrier()

        def _issue(ch, b):
            off = base + ch * K
            pltpu.async_copy(e_hbm.at[pl.ds(off, K), :], rows.at[b], sr[b])
            pltpu.async_copy(dst_hbm.at[pl.ds(off, K)], ix[b], sx[b])

        def _step(ch, b):
            pltpu.make_async_copy(e_hbm.at[pl.ds(base, K), :], rows.at[b],
                                  sr[b]).wait()
            pltpu.make_async_copy(dst_hbm.at[pl.ds(base, K)], ix[b],
                                  sx[b]).wait()

            pltpu.sync_copy(rows.at[b], acc_sh.at[ix[b]], add=True)

            @pl.when(ch + 2 < NCHUNK)
            def _():
                _issue(ch + 2, b)

        _issue(0, 0)
        _issue(1, 1)

        @pl.loop(0, NCHUNK - 1, step=2)
        def _pair(ci):
            _step(ci, 0)
            _step(ci + 1, 1)

        _step(NCHUNK - 1, 0)

        if TAIL:
            toff = base + NCHUNK * K
            pltpu.sync_copy(dst_hbm.at[pl.ds(toff, TAIL)], ixt)
            pltpu.sync_copy(e_hbm.at[pl.ds(toff, TAIL), :],
                            rows.at[0, pl.ds(0, TAIL), :])
            pltpu.sync_copy(rows.at[0, pl.ds(0, TAIL), :], acc_sh.at[ixt],
                            add=True)

        plsc.subcore_barrier()
        pltpu.sync_copy(acc_sh.at[pl.ds(s * RPS, RPS), :],
                        out_hbm.at[pl.ds(c * NP + s * RPS, RPS), :])

    return _sc_scatter_add


_sc_gather_half = _make_sc_gather(E2, 128)
_sc_scatter_half = _make_sc_scatter(E2, 128)


def _proj_body(x_ref, wa_ref, wb_ref, u_ref, v_ref):
    x = x_ref[...]
    u_ref[...] = jnp.dot(x, wa_ref[...], preferred_element_type=jnp.float32)
    v_ref[...] = jnp.dot(x, wb_ref[...], preferred_element_type=jnp.float32)


def _edge_core(g_ref, ef_ref, wef_ref, b1_ref, w2_ref, b2_ref, w3_ref, b3_ref,
               lng_ref, lnb_ref):
    ef = ef_ref[...]
    t = g_ref[...] + jnp.dot(
        ef.astype(jnp.bfloat16), wef_ref[...],
        preferred_element_type=jnp.float32) + b1_ref[...]
    t = jnp.maximum(t, 0.0)
    t = jnp.dot(t.astype(jnp.bfloat16), w2_ref[...],
                preferred_element_type=jnp.float32) + b2_ref[...]
    t = jnp.maximum(t, 0.0)
    e = jnp.dot(t.astype(jnp.bfloat16), w3_ref[...],
                preferred_element_type=jnp.float32) + b3_ref[...]
    m = jnp.mean(e, axis=-1, keepdims=True)
    var = jnp.mean((e - m) ** 2, axis=-1, keepdims=True)
    return (e - m) * lax.rsqrt(var + 1e-5) * lng_ref[...] + lnb_ref[...], ef


def _edge_body(g_ref, ef_ref, wef_ref, b1_ref, w2_ref, b2_ref, w3_ref, b3_ref,
               lng_ref, lnb_ref, e_ref, efo_ref):
    e, ef = _edge_core(g_ref, ef_ref, wef_ref, b1_ref, w2_ref, b2_ref, w3_ref,
                       b3_ref, lng_ref, lnb_ref)
    e_ref[...] = e
    efo_ref[...] = (e + ef.astype(jnp.float32)).astype(jnp.bfloat16)


def _edge_body_last(g_ref, ef_ref, wef_ref, b1_ref, w2_ref, b2_ref, w3_ref,
                    b3_ref, lng_ref, lnb_ref, e_ref):
    e, _ = _edge_core(g_ref, ef_ref, wef_ref, b1_ref, w2_ref, b2_ref, w3_ref,
                      b3_ref, lng_ref, lnb_ref)
    e_ref[...] = e


def _node_body(x_ref, a0_ref, a1_ref, a2_ref, a3_ref, wx_ref, wa_ref, b1_ref,
               w2_ref, b2_ref, w3_ref, b3_ref, lng_ref, lnb_ref, o_ref):
    x = x_ref[...]
    agg = (a0_ref[...] + a1_ref[...]) + (a2_ref[...] + a3_ref[...])
    t = (jnp.dot(x, wx_ref[...], preferred_element_type=jnp.float32)
         + jnp.dot(agg, wa_ref[...], preferred_element_type=jnp.float32)
         + b1_ref[...])
    t = jnp.maximum(t, 0.0)
    t = jnp.dot(t, w2_ref[...], preferred_element_type=jnp.float32) + b2_ref[...]
    t = jnp.maximum(t, 0.0)
    nx = jnp.dot(t, w3_ref[...], preferred_element_type=jnp.float32) + b3_ref[...]
    m = jnp.mean(nx, axis=-1, keepdims=True)
    var = jnp.mean((nx - m) ** 2, axis=-1, keepdims=True)
    nx = (nx - m) * lax.rsqrt(var + 1e-5) * lng_ref[...] + lnb_ref[...]
    o_ref[...] = nx + x


def _full(shape):
    nd = len(shape)
    return pl.BlockSpec(shape, lambda i: (0,) * nd)


def _rows(block, off=0):
    return pl.BlockSpec((block, D), lambda i, off=off: (i + off, 0))


def _tc_proj(x, wa, wb):
    return pl.pallas_call(
        _proj_body,
        grid=(N // BN,),
        in_specs=[_rows(BN), _full((D, D)), _full((D, D))],
        out_specs=[_rows(BN), _rows(BN)],
        out_shape=[jax.ShapeDtypeStruct((N, D), jnp.float32)] * 2,
    )(x, wa, wb)


def _tc_edge(g, ef, wef, b1, w2, b2, w3, b3, lng, lnb, ef_blk_off, last):
    in_specs = [_rows(BE), _rows(BE, ef_blk_off), _full((D, D)), _full((1, D)),
                _full((D, D)), _full((1, D)), _full((D, D)), _full((1, D)),
                _full((1, D)), _full((1, D))]
    if last:
        return pl.pallas_call(
            _edge_body_last,
            grid=(E2 // BE,),
            in_specs=in_specs,
            out_specs=_rows(BE),
            out_shape=jax.ShapeDtypeStruct((E2, D), jnp.float32),
        )(g, ef, wef, b1, w2, b2, w3, b3, lng, lnb), None
    e, efo = pl.pallas_call(
        _edge_body,
        grid=(E2 // BE,),
        in_specs=in_specs,
        out_specs=[_rows(BE), _rows(BE)],
        out_shape=[jax.ShapeDtypeStruct((E2, D), jnp.float32),
                   jax.ShapeDtypeStruct((E2, D), jnp.bfloat16)],
    )(g, ef, wef, b1, w2, b2, w3, b3, lng, lnb)
    return e, efo


def _tc_node(x, pa, pb, wx, wa, b1, w2, b2, w3, b3, lng, lnb):
    return pl.pallas_call(
        _node_body,
        grid=(N // BN,),
        in_specs=[_rows(BN), _rows(BN), _rows(BN), _rows(BN), _rows(BN),
                  _full((D, D)), _full((D, D)), _full((1, D)), _full((D, D)),
                  _full((1, D)), _full((D, D)), _full((1, D)), _full((1, D)),
                  _full((1, D))],
        out_specs=_rows(BN),
        out_shape=jax.ShapeDtypeStruct((N, D), jnp.float32),
    )(x, pa[:N], pa[NP:NP + N], pb[:N], pb[NP:NP + N],
      wx, wa, b1, w2, b2, w3, b3, lng, lnb)


def _row2(b):
    return jnp.reshape(b, (1, D))


def kernel(x, edge_index, edge_features, params):
    src = edge_index[0].astype(jnp.int32)
    dst = edge_index[1].astype(jnp.int32)
    src_h = (src[:E2], src[E2:])
    dst_h = (dst[:E2], dst[E2:])
    ef_h = (edge_features, edge_features)
    ef_off = (0, E2 // BE)
    nsteps = len(params)
    for si, p in enumerate(params):
        (w1, b1), (w2, b2), (w3, b3) = p["edge_mlp"]
        lng_e, lnb_e = p["edge_ln"]
        (nw1, nb1), (nw2, nb2), (nw3, nb3) = p["node_mlp"]
        lng_n, lnb_n = p["node_ln"]
        last = si == nsteps - 1

        u, v = _tc_proj(x, w1[:D], w1[D:2 * D])
        g0 = _sc_gather_half(u, v, src_h[0], dst_h[0])
        g1 = _sc_gather_half(u, v, src_h[1], dst_h[1])
        wef_bf = w1[2 * D:].astype(jnp.bfloat16)
        w2_bf = w2.astype(jnp.bfloat16)
        w3_bf = w3.astype(jnp.bfloat16)
        e0, efo0 = _tc_edge(g0, ef_h[0], wef_bf, _row2(b1), w2_bf, _row2(b2),
                            w3_bf, _row2(b3), _row2(lng_e), _row2(lnb_e),
                            ef_off[0], last)
        p0 = _sc_scatter_half(e0, dst_h[0])
        e1, efo1 = _tc_edge(g1, ef_h[1], wef_bf, _row2(b1), w2_bf, _row2(b2),
                            w3_bf, _row2(b3), _row2(lng_e), _row2(lnb_e),
                            ef_off[1], last)
        p1 = _sc_scatter_half(e1, dst_h[1])
        x = _tc_node(x, p0, p1, nw1[:D], nw1[D:], _row2(nb1), nw2, _row2(nb2),
                     nw3, _row2(nb3), _row2(lng_n), _row2(lnb_n))
        ef_h = (efo0, efo1)
        ef_off = (0, 0)
    return x


# speedup vs baseline: 5.8505x; 1.0010x over previous
import functools

import jax
import jax.numpy as jnp
from jax import lax
from jax.experimental import pallas as pl
from jax.experimental.pallas import tpu as pltpu
from jax.experimental.pallas import tpu_sc as plsc

N = 10000
E = 320000
D = 128
E2 = E // 2

NC = 2
NS = 16
NW = NC * NS
NP = 10240
RPS = NP // NS
ZR = 64

BN = 2000
BE = 2000

_MESH = plsc.VectorSubcoreMesh(core_axis_name="c", subcore_axis_name="s")


def _make_sc_gather(EH, K):
    EPW = EH // NW
    NCHUNK = EPW // K
    TAIL = EPW - NCHUNK * K
    assert NCHUNK % 2 == 1 and K % 8 == 0 and K <= 128 and EPW % 8 == 0
    assert TAIL % 8 == 0 and (NCHUNK * K) % 8 == 0

    @functools.partial(
        pl.kernel,
        out_type=jax.ShapeDtypeStruct((EH, D), jnp.float32),
        mesh=_MESH,
        scratch_types=[
            pltpu.VMEM((EPW,), jnp.int32),
            pltpu.VMEM((EPW,), jnp.int32),
            pltpu.VMEM((2, K, D), jnp.float32),
            pltpu.VMEM((2, K, D), jnp.float32),
            pltpu.VMEM((2, K, D), jnp.float32),
            pltpu.SemaphoreType.DMA,
            pltpu.SemaphoreType.DMA,
            pltpu.SemaphoreType.DMA,
            pltpu.SemaphoreType.DMA,
            pltpu.SemaphoreType.DMA,
            pltpu.SemaphoreType.DMA,
        ],
    )
    def _sc_gather_uv(u_hbm, v_hbm, src_hbm, dst_hbm, out_hbm,
                      si, di, ub, vb, ob, su0, su1, sv0, sv1, so0, so1):
        c = lax.axis_index("c")
        s = lax.axis_index("s")
        wid = s * NC + c
        base = wid * EPW
        su = (su0, su1)
        sv = (sv0, sv1)
        so = (so0, so1)

        pltpu.sync_copy(src_hbm.at[pl.ds(base, EPW)], si)
        pltpu.sync_copy(dst_hbm.at[pl.ds(base, EPW)], di)

        def _issue(ch, b):
            pltpu.async_copy(u_hbm.at[si.at[pl.ds(ch * K, K)]], ub.at[b], su[b])
            pltpu.async_copy(v_hbm.at[di.at[pl.ds(ch * K, K)]], vb.at[b], sv[b])

        def _wait_gather(b):
            pltpu.make_async_copy(u_hbm.at[si.at[pl.ds(0, K)]], ub.at[b],
                                  su[b]).wait()
            pltpu.make_async_copy(v_hbm.at[di.at[pl.ds(0, K)]], vb.at[b],
                                  sv[b]).wait()

        def _wait_out(b):
            pltpu.make_async_copy(ob.at[b], out_hbm.at[pl.ds(base, K), :],
                                  so[b]).wait()

        def _add_rows(b, nrows):
            @pl.loop(0, nrows)
            def _row(i):
                @pl.loop(0, D // 16, unroll=8)
                def _col(j):
                    sl = pl.ds(j * 16, 16)
                    ob[b, i, sl] = ub[b, i, sl] + vb[b, i, sl]

        def _step(ch, b):
            _wait_gather(b)

            @pl.when(ch >= 2)
            def _():
                _wait_out(b)

            _add_rows(b, K)
            pltpu.async_copy(ob.at[b], out_hbm.at[pl.ds(base + ch * K, K), :],
                             so[b])

            @pl.when(ch + 2 < NCHUNK)
            def _():
                _issue(ch + 2, b)

        _issue(0, 0)
        _issue(1, 1)

        @pl.loop(0, NCHUNK - 1, step=2)
        def _pair(ci):
            _step(ci, 0)
            _step(ci + 1, 1)

        _step(NCHUNK - 1, 0)
        _wait_out(1)
        _wait_out(0)

        if TAIL:
            toff = NCHUNK * K
            pltpu.async_copy(u_hbm.at[si.at[pl.ds(toff, TAIL)]],
                             ub.at[0, pl.ds(0, TAIL), :], su[0])
            pltpu.async_copy(v_hbm.at[di.at[pl.ds(toff, TAIL)]],
                             vb.at[0, pl.ds(0, TAIL), :], sv[0])
            pltpu.make_async_copy(u_hbm.at[si.at[pl.ds(0, TAIL)]],
                                  ub.at[0, pl.ds(0, TAIL), :], su[0]).wait()
            pltpu.make_async_copy(v_hbm.at[di.at[pl.ds(0, TAIL)]],
                                  vb.at[0, pl.ds(0, TAIL), :], sv[0]).wait()
            _add_rows(0, TAIL)
            pltpu.sync_copy(ob.at[0, pl.ds(0, TAIL), :],
                            out_hbm.at[pl.ds(base + toff, TAIL), :])

    return _sc_gather_uv


def _make_sc_scatter(EH, K):
    EPW = EH // NW
    NCHUNK = EPW // K
    TAIL = EPW - NCHUNK * K
    assert NCHUNK % 2 == 1 and K % 8 == 0 and K <= 128 and EPW % 8 == 0
    assert TAIL % 8 == 0 and (NCHUNK * K) % 8 == 0

    @functools.partial(
        pl.kernel,
        out_type=jax.ShapeDtypeStruct((2 * NP, D), jnp.float32),
        mesh=_MESH,
        scratch_types=[
            pltpu.VMEM((K,), jnp.int32),
            pltpu.VMEM((K,), jnp.int32),
            pltpu.VMEM((max(TAIL, 8),), jnp.int32),
            pltpu.VMEM((2, K, D), jnp.float32),
            pltpu.VMEM((ZR, D), jnp.float32),
            pltpu.VMEM_SHARED((NP, D), jnp.float32),
            pltpu.SemaphoreType.DMA,
            pltpu.SemaphoreType.DMA,
            pltpu.SemaphoreType.DMA,
            pltpu.SemaphoreType.DMA,
        ],
    )
    def _sc_scatter_add(e_hbm, dst_hbm, out_hbm, ix0, ix1, ixt, rows, zb_v,
                        acc_sh, sr0, sr1, sx0, sx1):
        c = lax.axis_index("c")
        s = lax.axis_index("s")
        wid = s * NC + c
        base = wid * EPW
        ix = (ix0, ix1)
        sr = (sr0, sr1)
        sx = (sx0, sx1)

        @pl.loop(0, ZR)
        def _zrow(i):
            @pl.loop(0, D // 16, unroll=8)
            def _zcol(j):
                zb_v[i, pl.ds(j * 16, 16)] = jnp.zeros((16,), jnp.float32)

        @pl.loop(0, RPS // ZR)
        def _zcopy(k):
            pltpu.sync_copy(zb_v, acc_sh.at[pl.ds(s * RPS + k * ZR, ZR), :])

        plsc.subcore_barrier()

        def _issue(ch, b):
            off = base + ch * K
            pltpu.async_copy(e_hbm.at[pl.ds(off, K), :], rows.at[b], sr[b])
            pltpu.async_copy(dst_hbm.at[pl.ds(off, K)], ix[b], sx[b])

        def _step(ch, b):
            pltpu.make_async_copy(e_hbm.at[pl.ds(base, K), :], rows.at[b],
                                  sr[b]).wait()
            pltpu.make_async_copy(dst_hbm.at[pl.ds(base, K)], ix[b],
                                  sx[b]).wait()

            pltpu.sync_copy(rows.at[b], acc_sh.at[ix[b]], add=True)

            @pl.when(ch + 2 < NCHUNK)
            def _():
                _issue(ch + 2, b)

        _issue(0, 0)
        _issue(1, 1)

        @pl.loop(0, NCHUNK - 1, step=2)
        def _pair(ci):
            _step(ci, 0)
            _step(ci + 1, 1)

        _step(NCHUNK - 1, 0)

        if TAIL:
            toff = base + NCHUNK * K
            pltpu.sync_copy(dst_hbm.at[pl.ds(toff, TAIL)], ixt)
            pltpu.sync_copy(e_hbm.at[pl.ds(toff, TAIL), :],
                            rows.at[0, pl.ds(0, TAIL), :])
            pltpu.sync_copy(rows.at[0, pl.ds(0, TAIL), :], acc_sh.at[ixt],
                            add=True)

        plsc.subcore_barrier()
        pltpu.sync_copy(acc_sh.at[pl.ds(s * RPS, RPS), :],
                        out_hbm.at[pl.ds(c * NP + s * RPS, RPS), :])

    return _sc_scatter_add


_sc_gather_half = _make_sc_gather(E2, 128)
_sc_scatter_half = _make_sc_scatter(E2, 128)


def _proj_body(x_ref, wa_ref, wb_ref, u_ref, v_ref):
    x = x_ref[...]
    u_ref[...] = jnp.dot(x, wa_ref[...], preferred_element_type=jnp.float32)
    v_ref[...] = jnp.dot(x, wb_ref[...], preferred_element_type=jnp.float32)


def _edge_core(g_ref, ef_ref, wef_ref, b1_ref, w2_ref, b2_ref, w3_ref, b3_ref,
               lng_ref, lnb_ref):
    ef = ef_ref[...]
    t = g_ref[...] + jnp.dot(
        ef.astype(jnp.bfloat16), wef_ref[...],
        preferred_element_type=jnp.float32) + b1_ref[...]
    t = jnp.maximum(t, 0.0)
    t = jnp.dot(t.astype(jnp.bfloat16), w2_ref[...],
                preferred_element_type=jnp.float32) + b2_ref[...]
    t = jnp.maximum(t, 0.0)
    e = jnp.dot(t.astype(jnp.bfloat16), w3_ref[...],
                preferred_element_type=jnp.float32) + b3_ref[...]
    m = jnp.mean(e, axis=-1, keepdims=True)
    var = jnp.mean((e - m) ** 2, axis=-1, keepdims=True)
    return (e - m) * lax.rsqrt(var + 1e-5) * lng_ref[...] + lnb_ref[...], ef


def _edge_body(g_ref, ef_ref, wef_ref, b1_ref, w2_ref, b2_ref, w3_ref, b3_ref,
               lng_ref, lnb_ref, e_ref, efo_ref):
    e, ef = _edge_core(g_ref, ef_ref, wef_ref, b1_ref, w2_ref, b2_ref, w3_ref,
                       b3_ref, lng_ref, lnb_ref)
    e_ref[...] = e
    efo_ref[...] = (e + ef.astype(jnp.float32)).astype(jnp.bfloat16)


def _edge_body_last(g_ref, ef_ref, wef_ref, b1_ref, w2_ref, b2_ref, w3_ref,
                    b3_ref, lng_ref, lnb_ref, e_ref):
    e, _ = _edge_core(g_ref, ef_ref, wef_ref, b1_ref, w2_ref, b2_ref, w3_ref,
                      b3_ref, lng_ref, lnb_ref)
    e_ref[...] = e


def _node_new_x(x_ref, a0_ref, a1_ref, a2_ref, a3_ref, wx_ref, wa_ref, b1_ref,
                w2_ref, b2_ref, w3_ref, b3_ref, lng_ref, lnb_ref):
    x = x_ref[...]
    agg = (a0_ref[...] + a1_ref[...]) + (a2_ref[...] + a3_ref[...])
    t = (jnp.dot(x, wx_ref[...], preferred_element_type=jnp.float32)
         + jnp.dot(agg, wa_ref[...], preferred_element_type=jnp.float32)
         + b1_ref[...])
    t = jnp.maximum(t, 0.0)
    t = jnp.dot(t, w2_ref[...], preferred_element_type=jnp.float32) + b2_ref[...]
    t = jnp.maximum(t, 0.0)
    nx = jnp.dot(t, w3_ref[...], preferred_element_type=jnp.float32) + b3_ref[...]
    m = jnp.mean(nx, axis=-1, keepdims=True)
    var = jnp.mean((nx - m) ** 2, axis=-1, keepdims=True)
    nx = (nx - m) * lax.rsqrt(var + 1e-5) * lng_ref[...] + lnb_ref[...]
    return nx + x


def _node_body(x_ref, a0_ref, a1_ref, a2_ref, a3_ref, wx_ref, wa_ref, b1_ref,
               w2_ref, b2_ref, w3_ref, b3_ref, lng_ref, lnb_ref, o_ref):
    o_ref[...] = _node_new_x(x_ref, a0_ref, a1_ref, a2_ref, a3_ref, wx_ref,
                             wa_ref, b1_ref, w2_ref, b2_ref, w3_ref, b3_ref,
                             lng_ref, lnb_ref)


def _node_body_proj(x_ref, a0_ref, a1_ref, a2_ref, a3_ref, wx_ref, wa_ref,
                    b1_ref, w2_ref, b2_ref, w3_ref, b3_ref, lng_ref, lnb_ref,
                    pwa_ref, pwb_ref, o_ref, u_ref, v_ref):
    xn = _node_new_x(x_ref, a0_ref, a1_ref, a2_ref, a3_ref, wx_ref, wa_ref,
                     b1_ref, w2_ref, b2_ref, w3_ref, b3_ref, lng_ref, lnb_ref)
    o_ref[...] = xn
    u_ref[...] = jnp.dot(xn, pwa_ref[...], preferred_element_type=jnp.float32)
    v_ref[...] = jnp.dot(xn, pwb_ref[...], preferred_element_type=jnp.float32)


def _full(shape):
    nd = len(shape)
    return pl.BlockSpec(shape, lambda i: (0,) * nd)


def _rows(block, off=0):
    return pl.BlockSpec((block, D), lambda i, off=off: (i + off, 0))


def _tc_proj(x, wa, wb):
    return pl.pallas_call(
        _proj_body,
        grid=(N // BN,),
        in_specs=[_rows(BN), _full((D, D)), _full((D, D))],
        out_specs=[_rows(BN), _rows(BN)],
        out_shape=[jax.ShapeDtypeStruct((N, D), jnp.float32)] * 2,
    )(x, wa, wb)


def _tc_edge(g, ef, wef, b1, w2, b2, w3, b3, lng, lnb, ef_blk_off, last):
    in_specs = [_rows(BE), _rows(BE, ef_blk_off), _full((D, D)), _full((1, D)),
                _full((D, D)), _full((1, D)), _full((D, D)), _full((1, D)),
                _full((1, D)), _full((1, D))]
    if last:
        return pl.pallas_call(
            _edge_body_last,
            grid=(E2 // BE,),
            in_specs=in_specs,
            out_specs=_rows(BE),
            out_shape=jax.ShapeDtypeStruct((E2, D), jnp.float32),
        )(g, ef, wef, b1, w2, b2, w3, b3, lng, lnb), None
    e, efo = pl.pallas_call(
        _edge_body,
        grid=(E2 // BE,),
        in_specs=in_specs,
        out_specs=[_rows(BE), _rows(BE)],
        out_shape=[jax.ShapeDtypeStruct((E2, D), jnp.float32),
                   jax.ShapeDtypeStruct((E2, D), jnp.bfloat16)],
    )(g, ef, wef, b1, w2, b2, w3, b3, lng, lnb)
    return e, efo


def _tc_node(x, pa, pb, wx, wa, b1, w2, b2, w3, b3, lng, lnb, next_w1=None):
    base_specs = [_rows(BN), _rows(BN), _rows(BN), _rows(BN), _rows(BN),
                  _full((D, D)), _full((D, D)), _full((1, D)), _full((D, D)),
                  _full((1, D)), _full((D, D)), _full((1, D)), _full((1, D)),
                  _full((1, D))]
    args = (x, pa[:N], pa[NP:NP + N], pb[:N], pb[NP:NP + N],
            wx, wa, b1, w2, b2, w3, b3, lng, lnb)
    if next_w1 is None:
        return pl.pallas_call(
            _node_body,
            grid=(N // BN,),
            in_specs=base_specs,
            out_specs=_rows(BN),
            out_shape=jax.ShapeDtypeStruct((N, D), jnp.float32),
        )(*args), None, None
    return pl.pallas_call(
        _node_body_proj,
        grid=(N // BN,),
        in_specs=base_specs + [_full((D, D)), _full((D, D))],
        out_specs=[_rows(BN), _rows(BN), _rows(BN)],
        out_shape=[jax.ShapeDtypeStruct((N, D), jnp.float32)] * 3,
    )(*args, next_w1[:D], next_w1[D:2 * D])


def _row2(b):
    return jnp.reshape(b, (1, D))


def kernel(x, edge_index, edge_features, params):
    src = edge_index[0].astype(jnp.int32)
    dst = edge_index[1].astype(jnp.int32)
    src_h = (src[:E2], src[E2:])
    dst_h = (dst[:E2], dst[E2:])
    ef_h = (edge_features, edge_features)
    ef_off = (0, E2 // BE)
    nsteps = len(params)
    u = v = None
    for si, p in enumerate(params):
        (w1, b1), (w2, b2), (w3, b3) = p["edge_mlp"]
        lng_e, lnb_e = p["edge_ln"]
        (nw1, nb1), (nw2, nb2), (nw3, nb3) = p["node_mlp"]
        lng_n, lnb_n = p["node_ln"]
        last = si == nsteps - 1

        if u is None:
            u, v = _tc_proj(x, w1[:D], w1[D:2 * D])
        g0 = _sc_gather_half(u, v, src_h[0], dst_h[0])
        g1 = _sc_gather_half(u, v, src_h[1], dst_h[1])
        wef_bf = w1[2 * D:].astype(jnp.bfloat16)
        w2_bf = w2.astype(jnp.bfloat16)
        w3_bf = w3.astype(jnp.bfloat16)
        e0, efo0 = _tc_edge(g0, ef_h[0], wef_bf, _row2(b1), w2_bf, _row2(b2),
                            w3_bf, _row2(b3), _row2(lng_e), _row2(lnb_e),
                            ef_off[0], last)
        p0 = _sc_scatter_half(e0, dst_h[0])
        e1, efo1 = _tc_edge(g1, ef_h[1], wef_bf, _row2(b1), w2_bf, _row2(b2),
                            w3_bf, _row2(b3), _row2(lng_e), _row2(lnb_e),
                            ef_off[1], last)
        p1 = _sc_scatter_half(e1, dst_h[1])
        next_w1 = None if last else params[si + 1]["edge_mlp"][0][0]
        x, u, v = _tc_node(x, p0, p1, nw1[:D], nw1[D:], _row2(nb1), nw2,
                           _row2(nb2), nw3, _row2(nb3), _row2(lng_n),
                           _row2(lnb_n), next_w1)
        ef_h = (efo0, efo1)
        ef_off = (0, 0)
    return x


# speedup vs baseline: 5.9400x; 1.0153x over previous
import functools

import jax
import jax.numpy as jnp
from jax import lax
from jax.experimental import pallas as pl
from jax.experimental.pallas import tpu as pltpu
from jax.experimental.pallas import tpu_sc as plsc

N = 10000
E = 320000
D = 128
E2 = E // 2

NC = 2
NS = 16
NW = NC * NS
NP = 10240
RPS = NP // NS
ZR = 64

BN = 2000
BE = 2000

_MESH = plsc.VectorSubcoreMesh(core_axis_name="c", subcore_axis_name="s")


def _make_sc_gather(EH, K):
    EPW = EH // NW
    NCHUNK = EPW // K
    TAIL = EPW - NCHUNK * K
    assert NCHUNK % 2 == 1 and K % 8 == 0 and K <= 128 and EPW % 8 == 0
    assert TAIL % 8 == 0 and (NCHUNK * K) % 8 == 0

    @functools.partial(
        pl.kernel,
        out_type=jax.ShapeDtypeStruct((EH, D), jnp.float32),
        mesh=_MESH,
        scratch_types=[
            pltpu.VMEM((EPW,), jnp.int32),
            pltpu.VMEM((EPW,), jnp.int32),
            pltpu.VMEM((2, K, D), jnp.float32),
            pltpu.VMEM((2, K, D), jnp.float32),
            pltpu.VMEM((2, K, D), jnp.float32),
            pltpu.VMEM_SHARED((N, D), jnp.float32),
            pltpu.SemaphoreType.DMA,
            pltpu.SemaphoreType.DMA,
            pltpu.SemaphoreType.DMA,
            pltpu.SemaphoreType.DMA,
            pltpu.SemaphoreType.DMA,
            pltpu.SemaphoreType.DMA,
        ],
    )
    def _sc_gather_uv(u_hbm, v_hbm, src_hbm, dst_hbm, out_hbm,
                      si, di, ub, vb, ob, vsh, su0, su1, sv0, sv1, so0, so1):
        c = lax.axis_index("c")
        s = lax.axis_index("s")
        wid = s * NC + c
        base = wid * EPW
        su = (su0, su1)
        sv = (sv0, sv1)
        so = (so0, so1)

        @pl.when(s < NS - 1)
        def _():
            pltpu.sync_copy(v_hbm.at[pl.ds(s * 640, 640), :],
                            vsh.at[pl.ds(s * 640, 640), :])

        @pl.when(s == NS - 1)
        def _():
            pltpu.sync_copy(v_hbm.at[pl.ds((NS - 1) * 640, N - (NS - 1) * 640), :],
                            vsh.at[pl.ds((NS - 1) * 640, N - (NS - 1) * 640), :])

        pltpu.sync_copy(src_hbm.at[pl.ds(base, EPW)], si)
        pltpu.sync_copy(dst_hbm.at[pl.ds(base, EPW)], di)
        plsc.subcore_barrier()

        def _issue(ch, b):
            pltpu.async_copy(u_hbm.at[si.at[pl.ds(ch * K, K)]], ub.at[b], su[b])
            pltpu.async_copy(vsh.at[di.at[pl.ds(ch * K, K)]], vb.at[b], sv[b])

        def _wait_gather(b):
            pltpu.make_async_copy(u_hbm.at[si.at[pl.ds(0, K)]], ub.at[b],
                                  su[b]).wait()
            pltpu.make_async_copy(vsh.at[di.at[pl.ds(0, K)]], vb.at[b],
                                  sv[b]).wait()

        def _wait_out(b):
            pltpu.make_async_copy(ob.at[b], out_hbm.at[pl.ds(base, K), :],
                                  so[b]).wait()

        def _add_rows(b, nrows):
            @pl.loop(0, nrows)
            def _row(i):
                @pl.loop(0, D // 16, unroll=8)
                def _col(j):
                    sl = pl.ds(j * 16, 16)
                    ob[b, i, sl] = ub[b, i, sl] + vb[b, i, sl]

        def _step(ch, b):
            _wait_gather(b)

            @pl.when(ch >= 2)
            def _():
                _wait_out(b)

            _add_rows(b, K)
            pltpu.async_copy(ob.at[b], out_hbm.at[pl.ds(base + ch * K, K), :],
                             so[b])

            @pl.when(ch + 2 < NCHUNK)
            def _():
                _issue(ch + 2, b)

        _issue(0, 0)
        _issue(1, 1)

        @pl.loop(0, NCHUNK - 1, step=2)
        def _pair(ci):
            _step(ci, 0)
            _step(ci + 1, 1)

        _step(NCHUNK - 1, 0)
        _wait_out(1)
        _wait_out(0)

        if TAIL:
            toff = NCHUNK * K
            pltpu.async_copy(u_hbm.at[si.at[pl.ds(toff, TAIL)]],
                             ub.at[0, pl.ds(0, TAIL), :], su[0])
            pltpu.async_copy(vsh.at[di.at[pl.ds(toff, TAIL)]],
                             vb.at[0, pl.ds(0, TAIL), :], sv[0])
            pltpu.make_async_copy(u_hbm.at[si.at[pl.ds(0, TAIL)]],
                                  ub.at[0, pl.ds(0, TAIL), :], su[0]).wait()
            pltpu.make_async_copy(vsh.at[di.at[pl.ds(0, TAIL)]],
                                  vb.at[0, pl.ds(0, TAIL), :], sv[0]).wait()
            _add_rows(0, TAIL)
            pltpu.sync_copy(ob.at[0, pl.ds(0, TAIL), :],
                            out_hbm.at[pl.ds(base + toff, TAIL), :])

    return _sc_gather_uv


def _make_sc_scatter(EH, K):
    EPW = EH // NW
    NCHUNK = EPW // K
    TAIL = EPW - NCHUNK * K
    assert NCHUNK % 2 == 1 and K % 8 == 0 and K <= 128 and EPW % 8 == 0
    assert TAIL % 8 == 0 and (NCHUNK * K) % 8 == 0

    @functools.partial(
        pl.kernel,
        out_type=jax.ShapeDtypeStruct((2 * NP, D), jnp.float32),
        mesh=_MESH,
        scratch_types=[
            pltpu.VMEM((K,), jnp.int32),
            pltpu.VMEM((K,), jnp.int32),
            pltpu.VMEM((max(TAIL, 8),), jnp.int32),
            pltpu.VMEM((2, K, D), jnp.float32),
            pltpu.VMEM((ZR, D), jnp.float32),
            pltpu.VMEM_SHARED((NP, D), jnp.float32),
            pltpu.SemaphoreType.DMA,
            pltpu.SemaphoreType.DMA,
            pltpu.SemaphoreType.DMA,
            pltpu.SemaphoreType.DMA,
        ],
    )
    def _sc_scatter_add(e_hbm, dst_hbm, out_hbm, ix0, ix1, ixt, rows, zb_v,
                        acc_sh, sr0, sr1, sx0, sx1):
        c = lax.axis_index("c")
        s = lax.axis_index("s")
        wid = s * NC + c
        base = wid * EPW
        ix = (ix0, ix1)
        sr = (sr0, sr1)
        sx = (sx0, sx1)

        @pl.loop(0, ZR)
        def _zrow(i):
            @pl.loop(0, D // 16, unroll=8)
            def _zcol(j):
                zb_v[i, pl.ds(j * 16, 16)] = jnp.zeros((16,), jnp.float32)

        @pl.loop(0, RPS // ZR)
        def _zcopy(k):
            pltpu.sync_copy(zb_v, acc_sh.at[pl.ds(s * RPS + k * ZR, ZR), :])

        plsc.subcore_barrier()

        def _issue(ch, b):
            off = base + ch * K
            pltpu.async_copy(e_hbm.at[pl.ds(off, K), :], rows.at[b], sr[b])
            pltpu.async_copy(dst_hbm.at[pl.ds(off, K)], ix[b], sx[b])

        def _step(ch, b):
            pltpu.make_async_copy(e_hbm.at[pl.ds(base, K), :], rows.at[b],
                                  sr[b]).wait()
            pltpu.make_async_copy(dst_hbm.at[pl.ds(base, K)], ix[b],
                                  sx[b]).wait()

            pltpu.sync_copy(rows.at[b], acc_sh.at[ix[b]], add=True)

            @pl.when(ch + 2 < NCHUNK)
            def _():
                _issue(ch + 2, b)

        _issue(0, 0)
        _issue(1, 1)

        @pl.loop(0, NCHUNK - 1, step=2)
        def _pair(ci):
            _step(ci, 0)
            _step(ci + 1, 1)

        _step(NCHUNK - 1, 0)

        if TAIL:
            toff = base + NCHUNK * K
            pltpu.sync_copy(dst_hbm.at[pl.ds(toff, TAIL)], ixt)
            pltpu.sync_copy(e_hbm.at[pl.ds(toff, TAIL), :],
                            rows.at[0, pl.ds(0, TAIL), :])
            pltpu.sync_copy(rows.at[0, pl.ds(0, TAIL), :], acc_sh.at[ixt],
                            add=True)

        plsc.subcore_barrier()
        pltpu.sync_copy(acc_sh.at[pl.ds(s * RPS, RPS), :],
                        out_hbm.at[pl.ds(c * NP + s * RPS, RPS), :])

    return _sc_scatter_add


_sc_gather_half = _make_sc_gather(E2, 40)
_sc_scatter_half = _make_sc_scatter(E2, 128)


def _proj_body(x_ref, wa_ref, wb_ref, u_ref, v_ref):
    x = x_ref[...]
    u_ref[...] = jnp.dot(x, wa_ref[...], preferred_element_type=jnp.float32)
    v_ref[...] = jnp.dot(x, wb_ref[...], preferred_element_type=jnp.float32)


def _edge_core(g_ref, ef_ref, wef_ref, b1_ref, w2_ref, b2_ref, w3_ref, b3_ref,
               lng_ref, lnb_ref):
    ef = ef_ref[...]
    t = g_ref[...] + jnp.dot(
        ef.astype(jnp.bfloat16), wef_ref[...],
        preferred_element_type=jnp.float32) + b1_ref[...]
    t = jnp.maximum(t, 0.0)
    t = jnp.dot(t.astype(jnp.bfloat16), w2_ref[...],
                preferred_element_type=jnp.float32) + b2_ref[...]
    t = jnp.maximum(t, 0.0)
    e = jnp.dot(t.astype(jnp.bfloat16), w3_ref[...],
                preferred_element_type=jnp.float32) + b3_ref[...]
    m = jnp.mean(e, axis=-1, keepdims=True)
    var = jnp.mean((e - m) ** 2, axis=-1, keepdims=True)
    return (e - m) * lax.rsqrt(var + 1e-5) * lng_ref[...] + lnb_ref[...], ef


def _edge_body(g_ref, ef_ref, wef_ref, b1_ref, w2_ref, b2_ref, w3_ref, b3_ref,
               lng_ref, lnb_ref, e_ref, efo_ref):
    e, ef = _edge_core(g_ref, ef_ref, wef_ref, b1_ref, w2_ref, b2_ref, w3_ref,
                       b3_ref, lng_ref, lnb_ref)
    e_ref[...] = e
    efo_ref[...] = (e + ef.astype(jnp.float32)).astype(jnp.bfloat16)


def _edge_body_last(g_ref, ef_ref, wef_ref, b1_ref, w2_ref, b2_ref, w3_ref,
                    b3_ref, lng_ref, lnb_ref, e_ref):
    e, _ = _edge_core(g_ref, ef_ref, wef_ref, b1_ref, w2_ref, b2_ref, w3_ref,
                      b3_ref, lng_ref, lnb_ref)
    e_ref[...] = e


def _node_new_x(x_ref, a0_ref, a1_ref, a2_ref, a3_ref, wx_ref, wa_ref, b1_ref,
                w2_ref, b2_ref, w3_ref, b3_ref, lng_ref, lnb_ref):
    x = x_ref[...]
    agg = (a0_ref[...] + a1_ref[...]) + (a2_ref[...] + a3_ref[...])
    t = (jnp.dot(x, wx_ref[...], preferred_element_type=jnp.float32)
         + jnp.dot(agg, wa_ref[...], preferred_element_type=jnp.float32)
         + b1_ref[...])
    t = jnp.maximum(t, 0.0)
    t = jnp.dot(t, w2_ref[...], preferred_element_type=jnp.float32) + b2_ref[...]
    t = jnp.maximum(t, 0.0)
    nx = jnp.dot(t, w3_ref[...], preferred_element_type=jnp.float32) + b3_ref[...]
    m = jnp.mean(nx, axis=-1, keepdims=True)
    var = jnp.mean((nx - m) ** 2, axis=-1, keepdims=True)
    nx = (nx - m) * lax.rsqrt(var + 1e-5) * lng_ref[...] + lnb_ref[...]
    return nx + x


def _node_body(x_ref, a0_ref, a1_ref, a2_ref, a3_ref, wx_ref, wa_ref, b1_ref,
               w2_ref, b2_ref, w3_ref, b3_ref, lng_ref, lnb_ref, o_ref):
    o_ref[...] = _node_new_x(x_ref, a0_ref, a1_ref, a2_ref, a3_ref, wx_ref,
                             wa_ref, b1_ref, w2_ref, b2_ref, w3_ref, b3_ref,
                             lng_ref, lnb_ref)


def _node_body_proj(x_ref, a0_ref, a1_ref, a2_ref, a3_ref, wx_ref, wa_ref,
                    b1_ref, w2_ref, b2_ref, w3_ref, b3_ref, lng_ref, lnb_ref,
                    pwa_ref, pwb_ref, o_ref, u_ref, v_ref):
    xn = _node_new_x(x_ref, a0_ref, a1_ref, a2_ref, a3_ref, wx_ref, wa_ref,
                     b1_ref, w2_ref, b2_ref, w3_ref, b3_ref, lng_ref, lnb_ref)
    o_ref[...] = xn
    u_ref[...] = jnp.dot(xn, pwa_ref[...], preferred_element_type=jnp.float32)
    v_ref[...] = jnp.dot(xn, pwb_ref[...], preferred_element_type=jnp.float32)


def _full(shape):
    nd = len(shape)
    return pl.BlockSpec(shape, lambda i: (0,) * nd)


def _rows(block, off=0):
    return pl.BlockSpec((block, D), lambda i, off=off: (i + off, 0))


def _tc_proj(x, wa, wb):
    return pl.pallas_call(
        _proj_body,
        grid=(N // BN,),
        in_specs=[_rows(BN), _full((D, D)), _full((D, D))],
        out_specs=[_rows(BN), _rows(BN)],
        out_shape=[jax.ShapeDtypeStruct((N, D), jnp.float32)] * 2,
    )(x, wa, wb)


def _tc_edge(g, ef, wef, b1, w2, b2, w3, b3, lng, lnb, ef_blk_off, last):
    in_specs = [_rows(BE), _rows(BE, ef_blk_off), _full((D, D)), _full((1, D)),
                _full((D, D)), _full((1, D)), _full((D, D)), _full((1, D)),
                _full((1, D)), _full((1, D))]
    if last:
        return pl.pallas_call(
            _edge_body_last,
            grid=(E2 // BE,),
            in_specs=in_specs,
            out_specs=_rows(BE),
            out_shape=jax.ShapeDtypeStruct((E2, D), jnp.float32),
        )(g, ef, wef, b1, w2, b2, w3, b3, lng, lnb), None
    e, efo = pl.pallas_call(
        _edge_body,
        grid=(E2 // BE,),
        in_specs=in_specs,
        out_specs=[_rows(BE), _rows(BE)],
        out_shape=[jax.ShapeDtypeStruct((E2, D), jnp.float32),
                   jax.ShapeDtypeStruct((E2, D), jnp.bfloat16)],
    )(g, ef, wef, b1, w2, b2, w3, b3, lng, lnb)
    return e, efo


def _tc_node(x, pa, pb, wx, wa, b1, w2, b2, w3, b3, lng, lnb, next_w1=None):
    base_specs = [_rows(BN), _rows(BN), _rows(BN), _rows(BN), _rows(BN),
                  _full((D, D)), _full((D, D)), _full((1, D)), _full((D, D)),
                  _full((1, D)), _full((D, D)), _full((1, D)), _full((1, D)),
                  _full((1, D))]
    args = (x, pa[:N], pa[NP:NP + N], pb[:N], pb[NP:NP + N],
            wx, wa, b1, w2, b2, w3, b3, lng, lnb)
    if next_w1 is None:
        return pl.pallas_call(
            _node_body,
            grid=(N // BN,),
            in_specs=base_specs,
            out_specs=_rows(BN),
            out_shape=jax.ShapeDtypeStruct((N, D), jnp.float32),
        )(*args), None, None
    return pl.pallas_call(
        _node_body_proj,
        grid=(N // BN,),
        in_specs=base_specs + [_full((D, D)), _full((D, D))],
        out_specs=[_rows(BN), _rows(BN), _rows(BN)],
        out_shape=[jax.ShapeDtypeStruct((N, D), jnp.float32)] * 3,
    )(*args, next_w1[:D], next_w1[D:2 * D])


def _row2(b):
    return jnp.reshape(b, (1, D))


def kernel(x, edge_index, edge_features, params):
    src = edge_index[0].astype(jnp.int32)
    dst = edge_index[1].astype(jnp.int32)
    src_h = (src[:E2], src[E2:])
    dst_h = (dst[:E2], dst[E2:])
    ef_h = (edge_features, edge_features)
    ef_off = (0, E2 // BE)
    nsteps = len(params)
    u = v = None
    for si, p in enumerate(params):
        (w1, b1), (w2, b2), (w3, b3) = p["edge_mlp"]
        lng_e, lnb_e = p["edge_ln"]
        (nw1, nb1), (nw2, nb2), (nw3, nb3) = p["node_mlp"]
        lng_n, lnb_n = p["node_ln"]
        last = si == nsteps - 1

        if u is None:
            u, v = _tc_proj(x, w1[:D], w1[D:2 * D])
        g0 = _sc_gather_half(u, v, src_h[0], dst_h[0])
        g1 = _sc_gather_half(u, v, src_h[1], dst_h[1])
        wef_bf = w1[2 * D:].astype(jnp.bfloat16)
        w2_bf = w2.astype(jnp.bfloat16)
        w3_bf = w3.astype(jnp.bfloat16)
        e0, efo0 = _tc_edge(g0, ef_h[0], wef_bf, _row2(b1), w2_bf, _row2(b2),
                            w3_bf, _row2(b3), _row2(lng_e), _row2(lnb_e),
                            ef_off[0], last)
        p0 = _sc_scatter_half(e0, dst_h[0])
        e1, efo1 = _tc_edge(g1, ef_h[1], wef_bf, _row2(b1), w2_bf, _row2(b2),
                            w3_bf, _row2(b3), _row2(lng_e), _row2(lnb_e),
                            ef_off[1], last)
        p1 = _sc_scatter_half(e1, dst_h[1])
        next_w1 = None if last else params[si + 1]["edge_mlp"][0][0]
        x, u, v = _tc_node(x, p0, p1, nw1[:D], nw1[D:], _row2(nb1), nw2,
                           _row2(nb2), nw3, _row2(nb3), _row2(lng_n),
                           _row2(lnb_n), next_w1)
        ef_h = (efo0, efo1)
        ef_off = (0, 0)
    return x


# speedup vs baseline: 6.0353x; 1.0160x over previous
import functools

import jax
import jax.numpy as jnp
from jax import lax
from jax.experimental import pallas as pl
from jax.experimental.pallas import tpu as pltpu
from jax.experimental.pallas import tpu_sc as plsc

N = 10000
E = 320000
D = 128
E2 = E // 2

NC = 2
NS = 16
NW = NC * NS
NP = 10240
RPS = NP // NS
ZR = 64

BN = 2000
BE = 2000

_MESH = plsc.VectorSubcoreMesh(core_axis_name="c", subcore_axis_name="s")


def _make_sc_gather(EH, K):
    EPW = EH // NW
    NCHUNK = EPW // K
    TAIL = EPW - NCHUNK * K
    assert K % 8 == 0 and K <= 128 and EPW % 8 == 0
    assert TAIL % 8 == 0 and (NCHUNK * K) % 8 == 0

    @functools.partial(
        pl.kernel,
        out_type=jax.ShapeDtypeStruct((EH, D), jnp.float32),
        mesh=_MESH,
        scratch_types=[
            pltpu.VMEM((EPW,), jnp.int32),
            pltpu.VMEM((EPW,), jnp.int32),
            pltpu.VMEM((2, K, D), jnp.float32),
            pltpu.VMEM((2, K, D), jnp.float32),
            pltpu.VMEM((2, K, D), jnp.float32),
            pltpu.VMEM_SHARED((N, D), jnp.float32),
            pltpu.SemaphoreType.DMA,
            pltpu.SemaphoreType.DMA,
            pltpu.SemaphoreType.DMA,
            pltpu.SemaphoreType.DMA,
            pltpu.SemaphoreType.DMA,
            pltpu.SemaphoreType.DMA,
        ],
    )
    def _sc_gather_uv(u_hbm, v_hbm, src_hbm, dst_hbm, out_hbm,
                      si, di, ub, vb, ob, vsh, su0, su1, sv0, sv1, so0, so1):
        c = lax.axis_index("c")
        s = lax.axis_index("s")
        wid = s * NC + c
        base = wid * EPW
        su = (su0, su1)
        sv = (sv0, sv1)
        so = (so0, so1)

        @pl.when(s < NS - 1)
        def _():
            pltpu.sync_copy(v_hbm.at[pl.ds(s * 640, 640), :],
                            vsh.at[pl.ds(s * 640, 640), :])

        @pl.when(s == NS - 1)
        def _():
            pltpu.sync_copy(v_hbm.at[pl.ds((NS - 1) * 640, N - (NS - 1) * 640), :],
                            vsh.at[pl.ds((NS - 1) * 640, N - (NS - 1) * 640), :])

        pltpu.sync_copy(src_hbm.at[pl.ds(base, EPW)], si)
        pltpu.sync_copy(dst_hbm.at[pl.ds(base, EPW)], di)
        plsc.subcore_barrier()

        def _issue(ch, b):
            pltpu.async_copy(u_hbm.at[si.at[pl.ds(ch * K, K)]], ub.at[b], su[b])
            pltpu.async_copy(vsh.at[di.at[pl.ds(ch * K, K)]], vb.at[b], sv[b])

        def _wait_gather(b):
            pltpu.make_async_copy(u_hbm.at[si.at[pl.ds(0, K)]], ub.at[b],
                                  su[b]).wait()
            pltpu.make_async_copy(vsh.at[di.at[pl.ds(0, K)]], vb.at[b],
                                  sv[b]).wait()

        def _wait_out(b):
            pltpu.make_async_copy(ob.at[b], out_hbm.at[pl.ds(base, K), :],
                                  so[b]).wait()

        def _add_rows(b, nrows):
            @pl.loop(0, nrows)
            def _row(i):
                @pl.loop(0, D // 16, unroll=8)
                def _col(j):
                    sl = pl.ds(j * 16, 16)
                    ob[b, i, sl] = ub[b, i, sl] + vb[b, i, sl]

        def _step(ch, b):
            _wait_gather(b)

            @pl.when(ch >= 2)
            def _():
                _wait_out(b)

            _add_rows(b, K)
            pltpu.async_copy(ob.at[b], out_hbm.at[pl.ds(base + ch * K, K), :],
                             so[b])

            @pl.when(ch + 2 < NCHUNK)
            def _():
                _issue(ch + 2, b)

        _issue(0, 0)
        _issue(1, 1)

        @pl.loop(0, NCHUNK - (NCHUNK % 2), step=2)
        def _pair(ci):
            _step(ci, 0)
            _step(ci + 1, 1)

        if NCHUNK % 2 == 1:
            _step(NCHUNK - 1, 0)
        _wait_out(1)
        _wait_out(0)

        if TAIL:
            toff = NCHUNK * K
            pltpu.async_copy(u_hbm.at[si.at[pl.ds(toff, TAIL)]],
                             ub.at[0, pl.ds(0, TAIL), :], su[0])
            pltpu.async_copy(vsh.at[di.at[pl.ds(toff, TAIL)]],
                             vb.at[0, pl.ds(0, TAIL), :], sv[0])
            pltpu.make_async_copy(u_hbm.at[si.at[pl.ds(0, TAIL)]],
                                  ub.at[0, pl.ds(0, TAIL), :], su[0]).wait()
            pltpu.make_async_copy(vsh.at[di.at[pl.ds(0, TAIL)]],
                                  vb.at[0, pl.ds(0, TAIL), :], sv[0]).wait()
            _add_rows(0, TAIL)
            pltpu.sync_copy(ob.at[0, pl.ds(0, TAIL), :],
                            out_hbm.at[pl.ds(base + toff, TAIL), :])

    return _sc_gather_uv


def _make_sc_scatter(EH, K):
    EPW = EH // NW
    NCHUNK = EPW // K
    TAIL = EPW - NCHUNK * K
    assert NCHUNK % 2 == 1 and K % 8 == 0 and K <= 128 and EPW % 8 == 0
    assert TAIL % 8 == 0 and (NCHUNK * K) % 8 == 0

    @functools.partial(
        pl.kernel,
        out_type=jax.ShapeDtypeStruct((2 * NP, D), jnp.float32),
        mesh=_MESH,
        scratch_types=[
            pltpu.VMEM((K,), jnp.int32),
            pltpu.VMEM((K,), jnp.int32),
            pltpu.VMEM((max(TAIL, 8),), jnp.int32),
            pltpu.VMEM((2, K, D), jnp.float32),
            pltpu.VMEM((ZR, D), jnp.float32),
            pltpu.VMEM_SHARED((NP, D), jnp.float32),
            pltpu.SemaphoreType.DMA,
            pltpu.SemaphoreType.DMA,
            pltpu.SemaphoreType.DMA,
            pltpu.SemaphoreType.DMA,
        ],
    )
    def _sc_scatter_add(e_hbm, dst_hbm, out_hbm, ix0, ix1, ixt, rows, zb_v,
                        acc_sh, sr0, sr1, sx0, sx1):
        c = lax.axis_index("c")
        s = lax.axis_index("s")
        wid = s * NC + c
        base = wid * EPW
        ix = (ix0, ix1)
        sr = (sr0, sr1)
        sx = (sx0, sx1)

        @pl.loop(0, ZR)
        def _zrow(i):
            @pl.loop(0, D // 16, unroll=8)
            def _zcol(j):
                zb_v[i, pl.ds(j * 16, 16)] = jnp.zeros((16,), jnp.float32)

        @pl.loop(0, RPS // ZR)
        def _zcopy(k):
            pltpu.sync_copy(zb_v, acc_sh.at[pl.ds(s * RPS + k * ZR, ZR), :])

        plsc.subcore_barrier()

        def _issue(ch, b):
            off = base + ch * K
            pltpu.async_copy(e_hbm.at[pl.ds(off, K), :], rows.at[b], sr[b])
            pltpu.async_copy(dst_hbm.at[pl.ds(off, K)], ix[b], sx[b])

        def _step(ch, b):
            pltpu.make_async_copy(e_hbm.at[pl.ds(base, K), :], rows.at[b],
                                  sr[b]).wait()
            pltpu.make_async_copy(dst_hbm.at[pl.ds(base, K)], ix[b],
                                  sx[b]).wait()

            pltpu.sync_copy(rows.at[b], acc_sh.at[ix[b]], add=True)

            @pl.when(ch + 2 < NCHUNK)
            def _():
                _issue(ch + 2, b)

        _issue(0, 0)
        _issue(1, 1)

        @pl.loop(0, NCHUNK - 1, step=2)
        def _pair(ci):
            _step(ci, 0)
            _step(ci + 1, 1)

        _step(NCHUNK - 1, 0)

        if TAIL:
            toff = base + NCHUNK * K
            pltpu.sync_copy(dst_hbm.at[pl.ds(toff, TAIL)], ixt)
            pltpu.sync_copy(e_hbm.at[pl.ds(toff, TAIL), :],
                            rows.at[0, pl.ds(0, TAIL), :])
            pltpu.sync_copy(rows.at[0, pl.ds(0, TAIL), :], acc_sh.at[ixt],
                            add=True)

        plsc.subcore_barrier()
        pltpu.sync_copy(acc_sh.at[pl.ds(s * RPS, RPS), :],
                        out_hbm.at[pl.ds(c * NP + s * RPS, RPS), :])

    return _sc_scatter_add


_sc_gather_half = _make_sc_gather(E2, 48)
_sc_scatter_half = _make_sc_scatter(E2, 128)


def _proj_body(x_ref, wa_ref, wb_ref, u_ref, v_ref):
    x = x_ref[...]
    u_ref[...] = jnp.dot(x, wa_ref[...], preferred_element_type=jnp.float32)
    v_ref[...] = jnp.dot(x, wb_ref[...], preferred_element_type=jnp.float32)


def _edge_core(g_ref, ef_ref, wef_ref, b1_ref, w2_ref, b2_ref, w3_ref, b3_ref,
               lng_ref, lnb_ref):
    ef = ef_ref[...]
    t = g_ref[...] + jnp.dot(
        ef.astype(jnp.bfloat16), wef_ref[...],
        preferred_element_type=jnp.float32) + b1_ref[...]
    t = jnp.maximum(t, 0.0)
    t = jnp.dot(t.astype(jnp.bfloat16), w2_ref[...],
                preferred_element_type=jnp.float32) + b2_ref[...]
    t = jnp.maximum(t, 0.0)
    e = jnp.dot(t.astype(jnp.bfloat16), w3_ref[...],
                preferred_element_type=jnp.float32) + b3_ref[...]
    m = jnp.mean(e, axis=-1, keepdims=True)
    var = jnp.mean((e - m) ** 2, axis=-1, keepdims=True)
    return (e - m) * lax.rsqrt(var + 1e-5) * lng_ref[...] + lnb_ref[...], ef


def _edge_body(g_ref, ef_ref, wef_ref, b1_ref, w2_ref, b2_ref, w3_ref, b3_ref,
               lng_ref, lnb_ref, e_ref, efo_ref):
    e, ef = _edge_core(g_ref, ef_ref, wef_ref, b1_ref, w2_ref, b2_ref, w3_ref,
                       b3_ref, lng_ref, lnb_ref)
    e_ref[...] = e
    efo_ref[...] = (e + ef.astype(jnp.float32)).astype(jnp.bfloat16)


def _edge_body_last(g_ref, ef_ref, wef_ref, b1_ref, w2_ref, b2_ref, w3_ref,
                    b3_ref, lng_ref, lnb_ref, e_ref):
    e, _ = _edge_core(g_ref, ef_ref, wef_ref, b1_ref, w2_ref, b2_ref, w3_ref,
                      b3_ref, lng_ref, lnb_ref)
    e_ref[...] = e


def _node_new_x(x_ref, a0_ref, a1_ref, a2_ref, a3_ref, wx_ref, wa_ref, b1_ref,
                w2_ref, b2_ref, w3_ref, b3_ref, lng_ref, lnb_ref):
    x = x_ref[...]
    agg = (a0_ref[...] + a1_ref[...]) + (a2_ref[...] + a3_ref[...])
    t = (jnp.dot(x, wx_ref[...], preferred_element_type=jnp.float32)
         + jnp.dot(agg, wa_ref[...], preferred_element_type=jnp.float32)
         + b1_ref[...])
    t = jnp.maximum(t, 0.0)
    t = jnp.dot(t, w2_ref[...], preferred_element_type=jnp.float32) + b2_ref[...]
    t = jnp.maximum(t, 0.0)
    nx = jnp.dot(t, w3_ref[...], preferred_element_type=jnp.float32) + b3_ref[...]
    m = jnp.mean(nx, axis=-1, keepdims=True)
    var = jnp.mean((nx - m) ** 2, axis=-1, keepdims=True)
    nx = (nx - m) * lax.rsqrt(var + 1e-5) * lng_ref[...] + lnb_ref[...]
    return nx + x


def _node_body(x_ref, a0_ref, a1_ref, a2_ref, a3_ref, wx_ref, wa_ref, b1_ref,
               w2_ref, b2_ref, w3_ref, b3_ref, lng_ref, lnb_ref, o_ref):
    o_ref[...] = _node_new_x(x_ref, a0_ref, a1_ref, a2_ref, a3_ref, wx_ref,
                             wa_ref, b1_ref, w2_ref, b2_ref, w3_ref, b3_ref,
                             lng_ref, lnb_ref)


def _node_body_proj(x_ref, a0_ref, a1_ref, a2_ref, a3_ref, wx_ref, wa_ref,
                    b1_ref, w2_ref, b2_ref, w3_ref, b3_ref, lng_ref, lnb_ref,
                    pwa_ref, pwb_ref, o_ref, u_ref, v_ref):
    xn = _node_new_x(x_ref, a0_ref, a1_ref, a2_ref, a3_ref, wx_ref, wa_ref,
                     b1_ref, w2_ref, b2_ref, w3_ref, b3_ref, lng_ref, lnb_ref)
    o_ref[...] = xn
    u_ref[...] = jnp.dot(xn, pwa_ref[...], preferred_element_type=jnp.float32)
    v_ref[...] = jnp.dot(xn, pwb_ref[...], preferred_element_type=jnp.float32)


def _full(shape):
    nd = len(shape)
    return pl.BlockSpec(shape, lambda i: (0,) * nd)


def _rows(block, off=0):
    return pl.BlockSpec((block, D), lambda i, off=off: (i + off, 0))


def _tc_proj(x, wa, wb):
    return pl.pallas_call(
        _proj_body,
        grid=(N // BN,),
        in_specs=[_rows(BN), _full((D, D)), _full((D, D))],
        out_specs=[_rows(BN), _rows(BN)],
        out_shape=[jax.ShapeDtypeStruct((N, D), jnp.float32)] * 2,
    )(x, wa, wb)


def _tc_edge(g, ef, wef, b1, w2, b2, w3, b3, lng, lnb, ef_blk_off, last):
    in_specs = [_rows(BE), _rows(BE, ef_blk_off), _full((D, D)), _full((1, D)),
                _full((D, D)), _full((1, D)), _full((D, D)), _full((1, D)),
                _full((1, D)), _full((1, D))]
    if last:
        return pl.pallas_call(
            _edge_body_last,
            grid=(E2 // BE,),
            in_specs=in_specs,
            out_specs=_rows(BE),
            out_shape=jax.ShapeDtypeStruct((E2, D), jnp.float32),
        )(g, ef, wef, b1, w2, b2, w3, b3, lng, lnb), None
    e, efo = pl.pallas_call(
        _edge_body,
        grid=(E2 // BE,),
        in_specs=in_specs,
        out_specs=[_rows(BE), _rows(BE)],
        out_shape=[jax.ShapeDtypeStruct((E2, D), jnp.float32),
                   jax.ShapeDtypeStruct((E2, D), jnp.bfloat16)],
    )(g, ef, wef, b1, w2, b2, w3, b3, lng, lnb)
    return e, efo


def _tc_node(x, pa, pb, wx, wa, b1, w2, b2, w3, b3, lng, lnb, next_w1=None):
    base_specs = [_rows(BN), _rows(BN), _rows(BN), _rows(BN), _rows(BN),
                  _full((D, D)), _full((D, D)), _full((1, D)), _full((D, D)),
                  _full((1, D)), _full((D, D)), _full((1, D)), _full((1, D)),
                  _full((1, D))]
    args = (x, pa[:N], pa[NP:NP + N], pb[:N], pb[NP:NP + N],
            wx, wa, b1, w2, b2, w3, b3, lng, lnb)
    if next_w1 is None:
        return pl.pallas_call(
            _node_body,
            grid=(N // BN,),
            in_specs=base_specs,
            out_specs=_rows(BN),
            out_shape=jax.ShapeDtypeStruct((N, D), jnp.float32),
        )(*args), None, None
    return pl.pallas_call(
        _node_body_proj,
        grid=(N // BN,),
        in_specs=base_specs + [_full((D, D)), _full((D, D))],
        out_specs=[_rows(BN), _rows(BN), _rows(BN)],
        out_shape=[jax.ShapeDtypeStruct((N, D), jnp.float32)] * 3,
    )(*args, next_w1[:D], next_w1[D:2 * D])


def _row2(b):
    return jnp.reshape(b, (1, D))


def kernel(x, edge_index, edge_features, params):
    src = edge_index[0].astype(jnp.int32)
    dst = edge_index[1].astype(jnp.int32)
    src_h = (src[:E2], src[E2:])
    dst_h = (dst[:E2], dst[E2:])
    ef_h = (edge_features, edge_features)
    ef_off = (0, E2 // BE)
    nsteps = len(params)
    u = v = None
    for si, p in enumerate(params):
        (w1, b1), (w2, b2), (w3, b3) = p["edge_mlp"]
        lng_e, lnb_e = p["edge_ln"]
        (nw1, nb1), (nw2, nb2), (nw3, nb3) = p["node_mlp"]
        lng_n, lnb_n = p["node_ln"]
        last = si == nsteps - 1

        if u is None:
            u, v = _tc_proj(x, w1[:D], w1[D:2 * D])
        g0 = _sc_gather_half(u, v, src_h[0], dst_h[0])
        g1 = _sc_gather_half(u, v, src_h[1], dst_h[1])
        wef_bf = w1[2 * D:].astype(jnp.bfloat16)
        w2_bf = w2.astype(jnp.bfloat16)
        w3_bf = w3.astype(jnp.bfloat16)
        e0, efo0 = _tc_edge(g0, ef_h[0], wef_bf, _row2(b1), w2_bf, _row2(b2),
                            w3_bf, _row2(b3), _row2(lng_e), _row2(lnb_e),
                            ef_off[0], last)
        p0 = _sc_scatter_half(e0, dst_h[0])
        e1, efo1 = _tc_edge(g1, ef_h[1], wef_bf, _row2(b1), w2_bf, _row2(b2),
                            w3_bf, _row2(b3), _row2(lng_e), _row2(lnb_e),
                            ef_off[1], last)
        p1 = _sc_scatter_half(e1, dst_h[1])
        next_w1 = None if last else params[si + 1]["edge_mlp"][0][0]
        x, u, v = _tc_node(x, p0, p1, nw1[:D], nw1[D:], _row2(nb1), nw2,
                           _row2(nb2), nw3, _row2(nb3), _row2(lng_n),
                           _row2(lnb_n), next_w1)
        ef_h = (efo0, efo1)
        ef_off = (0, 0)
    return x


# speedup vs baseline: 6.4832x; 1.0742x over previous
import functools

import jax
import jax.numpy as jnp
from jax import lax
from jax.experimental import pallas as pl
from jax.experimental.pallas import tpu as pltpu
from jax.experimental.pallas import tpu_sc as plsc

N = 10000
E = 320000
D = 128
E2 = E // 2

NC = 2
NS = 16
NW = NC * NS
NP = 10240
RPS = NP // NS
ZR = 64

BN = 2000
BE = 4000

_MESH = plsc.VectorSubcoreMesh(core_axis_name="c", subcore_axis_name="s")


def _make_sc_gather(EH, K):
    EPW = EH // NW
    NCHUNK = EPW // K
    TAIL = EPW - NCHUNK * K
    assert K % 8 == 0 and K <= 128 and EPW % 8 == 0
    assert TAIL % 8 == 0 and (NCHUNK * K) % 8 == 0

    @functools.partial(
        pl.kernel,
        out_type=jax.ShapeDtypeStruct((EH, D), jnp.float32),
        mesh=_MESH,
        scratch_types=[
            pltpu.VMEM((EPW,), jnp.int32),
            pltpu.VMEM((EPW,), jnp.int32),
            pltpu.VMEM((2, K, D), jnp.float32),
            pltpu.VMEM((2, K, D), jnp.float32),
            pltpu.VMEM((2, K, D), jnp.float32),
            pltpu.VMEM_SHARED((N, D), jnp.float32),
            pltpu.SemaphoreType.DMA,
            pltpu.SemaphoreType.DMA,
            pltpu.SemaphoreType.DMA,
            pltpu.SemaphoreType.DMA,
            pltpu.SemaphoreType.DMA,
            pltpu.SemaphoreType.DMA,
        ],
    )
    def _sc_gather_uv(u_hbm, v_hbm, src_hbm, dst_hbm, out_hbm,
                      si, di, ub, vb, ob, vsh, su0, su1, sv0, sv1, so0, so1):
        c = lax.axis_index("c")
        s = lax.axis_index("s")
        wid = s * NC + c
        base = wid * EPW
        su = (su0, su1)
        sv = (sv0, sv1)
        so = (so0, so1)

        @pl.when(s < NS - 1)
        def _():
            pltpu.sync_copy(v_hbm.at[pl.ds(s * 640, 640), :],
                            vsh.at[pl.ds(s * 640, 640), :])

        @pl.when(s == NS - 1)
        def _():
            pltpu.sync_copy(v_hbm.at[pl.ds((NS - 1) * 640, N - (NS - 1) * 640), :],
                            vsh.at[pl.ds((NS - 1) * 640, N - (NS - 1) * 640), :])

        pltpu.sync_copy(src_hbm.at[pl.ds(base, EPW)], si)
        pltpu.sync_copy(dst_hbm.at[pl.ds(base, EPW)], di)
        plsc.subcore_barrier()

        def _issue(ch, b):
            pltpu.async_copy(u_hbm.at[si.at[pl.ds(ch * K, K)]], ub.at[b], su[b])
            pltpu.async_copy(vsh.at[di.at[pl.ds(ch * K, K)]], vb.at[b], sv[b])

        def _wait_gather(b):
            pltpu.make_async_copy(u_hbm.at[si.at[pl.ds(0, K)]], ub.at[b],
                                  su[b]).wait()
            pltpu.make_async_copy(vsh.at[di.at[pl.ds(0, K)]], vb.at[b],
                                  sv[b]).wait()

        def _wait_out(b):
            pltpu.make_async_copy(ob.at[b], out_hbm.at[pl.ds(base, K), :],
                                  so[b]).wait()

        def _add_rows(b, nrows):
            @pl.loop(0, nrows)
            def _row(i):
                @pl.loop(0, D // 16, unroll=8)
                def _col(j):
                    sl = pl.ds(j * 16, 16)
                    ob[b, i, sl] = ub[b, i, sl] + vb[b, i, sl]

        def _step(ch, b):
            _wait_gather(b)

            @pl.when(ch >= 2)
            def _():
                _wait_out(b)

            _add_rows(b, K)
            pltpu.async_copy(ob.at[b], out_hbm.at[pl.ds(base + ch * K, K), :],
                             so[b])

            @pl.when(ch + 2 < NCHUNK)
            def _():
                _issue(ch + 2, b)

        _issue(0, 0)
        _issue(1, 1)

        @pl.loop(0, NCHUNK - (NCHUNK % 2), step=2)
        def _pair(ci):
            _step(ci, 0)
            _step(ci + 1, 1)

        if NCHUNK % 2 == 1:
            _step(NCHUNK - 1, 0)
        _wait_out(1)
        _wait_out(0)

        if TAIL:
            toff = NCHUNK * K
            pltpu.async_copy(u_hbm.at[si.at[pl.ds(toff, TAIL)]],
                             ub.at[0, pl.ds(0, TAIL), :], su[0])
            pltpu.async_copy(vsh.at[di.at[pl.ds(toff, TAIL)]],
                             vb.at[0, pl.ds(0, TAIL), :], sv[0])
            pltpu.make_async_copy(u_hbm.at[si.at[pl.ds(0, TAIL)]],
                                  ub.at[0, pl.ds(0, TAIL), :], su[0]).wait()
            pltpu.make_async_copy(vsh.at[di.at[pl.ds(0, TAIL)]],
                                  vb.at[0, pl.ds(0, TAIL), :], sv[0]).wait()
            _add_rows(0, TAIL)
            pltpu.sync_copy(ob.at[0, pl.ds(0, TAIL), :],
                            out_hbm.at[pl.ds(base + toff, TAIL), :])

    return _sc_gather_uv


def _make_sc_scatter(EH, K):
    EPW = EH // NW
    NCHUNK = EPW // K
    TAIL = EPW - NCHUNK * K
    assert NCHUNK % 2 == 1 and K % 8 == 0 and K <= 128 and EPW % 8 == 0
    assert TAIL % 8 == 0 and (NCHUNK * K) % 8 == 0

    @functools.partial(
        pl.kernel,
        out_type=jax.ShapeDtypeStruct((2 * NP, D), jnp.float32),
        mesh=_MESH,
        scratch_types=[
            pltpu.VMEM((K,), jnp.int32),
            pltpu.VMEM((K,), jnp.int32),
            pltpu.VMEM((max(TAIL, 8),), jnp.int32),
            pltpu.VMEM((2, K, D), jnp.float32),
            pltpu.VMEM((ZR, D), jnp.float32),
            pltpu.VMEM_SHARED((NP, D), jnp.float32),
            pltpu.SemaphoreType.DMA,
            pltpu.SemaphoreType.DMA,
            pltpu.SemaphoreType.DMA,
            pltpu.SemaphoreType.DMA,
        ],
    )
    def _sc_scatter_add(e_hbm, dst_hbm, out_hbm, ix0, ix1, ixt, rows, zb_v,
                        acc_sh, sr0, sr1, sx0, sx1):
        c = lax.axis_index("c")
        s = lax.axis_index("s")
        wid = s * NC + c
        base = wid * EPW
        ix = (ix0, ix1)
        sr = (sr0, sr1)
        sx = (sx0, sx1)

        @pl.loop(0, ZR)
        def _zrow(i):
            @pl.loop(0, D // 16, unroll=8)
            def _zcol(j):
                zb_v[i, pl.ds(j * 16, 16)] = jnp.zeros((16,), jnp.float32)

        @pl.loop(0, RPS // ZR)
        def _zcopy(k):
            pltpu.sync_copy(zb_v, acc_sh.at[pl.ds(s * RPS + k * ZR, ZR), :])

        plsc.subcore_barrier()

        def _issue(ch, b):
            off = base + ch * K
            pltpu.async_copy(e_hbm.at[pl.ds(off, K), :], rows.at[b], sr[b])
            pltpu.async_copy(dst_hbm.at[pl.ds(off, K)], ix[b], sx[b])

        def _step(ch, b):
            pltpu.make_async_copy(e_hbm.at[pl.ds(base, K), :], rows.at[b],
                                  sr[b]).wait()
            pltpu.make_async_copy(dst_hbm.at[pl.ds(base, K)], ix[b],
                                  sx[b]).wait()

            pltpu.sync_copy(rows.at[b], acc_sh.at[ix[b]], add=True)

            @pl.when(ch + 2 < NCHUNK)
            def _():
                _issue(ch + 2, b)

        _issue(0, 0)
        _issue(1, 1)

        @pl.loop(0, NCHUNK - 1, step=2)
        def _pair(ci):
            _step(ci, 0)
            _step(ci + 1, 1)

        _step(NCHUNK - 1, 0)

        if TAIL:
            toff = base + NCHUNK * K
            pltpu.sync_copy(dst_hbm.at[pl.ds(toff, TAIL)], ixt)
            pltpu.sync_copy(e_hbm.at[pl.ds(toff, TAIL), :],
                            rows.at[0, pl.ds(0, TAIL), :])
            pltpu.sync_copy(rows.at[0, pl.ds(0, TAIL), :], acc_sh.at[ixt],
                            add=True)

        plsc.subcore_barrier()
        pltpu.sync_copy(acc_sh.at[pl.ds(s * RPS, RPS), :],
                        out_hbm.at[pl.ds(c * NP + s * RPS, RPS), :])

    return _sc_scatter_add


_sc_gather_half = _make_sc_gather(E2, 48)
_sc_scatter_half = _make_sc_scatter(E2, 128)


def _proj_body(x_ref, wa_ref, wb_ref, u_ref, v_ref):
    x = x_ref[...]
    u_ref[...] = jnp.dot(x, wa_ref[...], preferred_element_type=jnp.float32)
    v_ref[...] = jnp.dot(x, wb_ref[...], preferred_element_type=jnp.float32)


def _edge_core(g_ref, ef_ref, wef_ref, b1_ref, w2_ref, b2_ref, w3_ref, b3_ref,
               lng_ref, lnb_ref):
    ef = ef_ref[...]
    t = g_ref[...] + jnp.dot(
        ef.astype(jnp.bfloat16), wef_ref[...],
        preferred_element_type=jnp.float32) + b1_ref[...]
    t = jnp.maximum(t, 0.0)
    t = jnp.dot(t.astype(jnp.bfloat16), w2_ref[...],
                preferred_element_type=jnp.float32) + b2_ref[...]
    t = jnp.maximum(t, 0.0)
    e = jnp.dot(t.astype(jnp.bfloat16), w3_ref[...],
                preferred_element_type=jnp.float32) + b3_ref[...]
    m = jnp.mean(e, axis=-1, keepdims=True)
    var = jnp.mean((e - m) ** 2, axis=-1, keepdims=True)
    return (e - m) * lax.rsqrt(var + 1e-5) * lng_ref[...] + lnb_ref[...], ef


def _edge_body(g_ref, ef_ref, wef_ref, b1_ref, w2_ref, b2_ref, w3_ref, b3_ref,
               lng_ref, lnb_ref, e_ref, efo_ref):
    e, ef = _edge_core(g_ref, ef_ref, wef_ref, b1_ref, w2_ref, b2_ref, w3_ref,
                       b3_ref, lng_ref, lnb_ref)
    e_ref[...] = e
    efo_ref[...] = (e + ef.astype(jnp.float32)).astype(jnp.bfloat16)


def _edge_body_last(g_ref, ef_ref, wef_ref, b1_ref, w2_ref, b2_ref, w3_ref,
                    b3_ref, lng_ref, lnb_ref, e_ref):
    e, _ = _edge_core(g_ref, ef_ref, wef_ref, b1_ref, w2_ref, b2_ref, w3_ref,
                      b3_ref, lng_ref, lnb_ref)
    e_ref[...] = e


def _node_new_x(x_ref, a0_ref, a1_ref, a2_ref, a3_ref, wx_ref, wa_ref, b1_ref,
                w2_ref, b2_ref, w3_ref, b3_ref, lng_ref, lnb_ref):
    x = x_ref[...]
    agg = (a0_ref[...] + a1_ref[...]) + (a2_ref[...] + a3_ref[...])
    t = (jnp.dot(x, wx_ref[...], preferred_element_type=jnp.float32)
         + jnp.dot(agg, wa_ref[...], preferred_element_type=jnp.float32)
         + b1_ref[...])
    t = jnp.maximum(t, 0.0)
    t = jnp.dot(t, w2_ref[...], preferred_element_type=jnp.float32) + b2_ref[...]
    t = jnp.maximum(t, 0.0)
    nx = jnp.dot(t, w3_ref[...], preferred_element_type=jnp.float32) + b3_ref[...]
    m = jnp.mean(nx, axis=-1, keepdims=True)
    var = jnp.mean((nx - m) ** 2, axis=-1, keepdims=True)
    nx = (nx - m) * lax.rsqrt(var + 1e-5) * lng_ref[...] + lnb_ref[...]
    return nx + x


def _node_body(x_ref, a0_ref, a1_ref, a2_ref, a3_ref, wx_ref, wa_ref, b1_ref,
               w2_ref, b2_ref, w3_ref, b3_ref, lng_ref, lnb_ref, o_ref):
    o_ref[...] = _node_new_x(x_ref, a0_ref, a1_ref, a2_ref, a3_ref, wx_ref,
                             wa_ref, b1_ref, w2_ref, b2_ref, w3_ref, b3_ref,
                             lng_ref, lnb_ref)


def _node_body_proj(x_ref, a0_ref, a1_ref, a2_ref, a3_ref, wx_ref, wa_ref,
                    b1_ref, w2_ref, b2_ref, w3_ref, b3_ref, lng_ref, lnb_ref,
                    pwa_ref, pwb_ref, o_ref, u_ref, v_ref):
    xn = _node_new_x(x_ref, a0_ref, a1_ref, a2_ref, a3_ref, wx_ref, wa_ref,
                     b1_ref, w2_ref, b2_ref, w3_ref, b3_ref, lng_ref, lnb_ref)
    o_ref[...] = xn
    u_ref[...] = jnp.dot(xn, pwa_ref[...], preferred_element_type=jnp.float32)
    v_ref[...] = jnp.dot(xn, pwb_ref[...], preferred_element_type=jnp.float32)


def _full(shape):
    nd = len(shape)
    return pl.BlockSpec(shape, lambda i: (0,) * nd)


def _rows(block, off=0):
    return pl.BlockSpec((block, D), lambda i, off=off: (i + off, 0))


def _tc_proj(x, wa, wb):
    return pl.pallas_call(
        _proj_body,
        grid=(N // BN,),
        in_specs=[_rows(BN), _full((D, D)), _full((D, D))],
        out_specs=[_rows(BN), _rows(BN)],
        out_shape=[jax.ShapeDtypeStruct((N, D), jnp.float32)] * 2,
    )(x, wa, wb)


def _tc_edge(g, ef, wef, b1, w2, b2, w3, b3, lng, lnb, ef_blk_off, last):
    in_specs = [_rows(BE), _rows(BE, ef_blk_off), _full((D, D)), _full((1, D)),
                _full((D, D)), _full((1, D)), _full((D, D)), _full((1, D)),
                _full((1, D)), _full((1, D))]
    if last:
        return pl.pallas_call(
            _edge_body_last,
            grid=(E2 // BE,),
            in_specs=in_specs,
            out_specs=_rows(BE),
            out_shape=jax.ShapeDtypeStruct((E2, D), jnp.float32),
        )(g, ef, wef, b1, w2, b2, w3, b3, lng, lnb), None
    e, efo = pl.pallas_call(
        _edge_body,
        grid=(E2 // BE,),
        in_specs=in_specs,
        out_specs=[_rows(BE), _rows(BE)],
        out_shape=[jax.ShapeDtypeStruct((E2, D), jnp.float32),
                   jax.ShapeDtypeStruct((E2, D), jnp.bfloat16)],
    )(g, ef, wef, b1, w2, b2, w3, b3, lng, lnb)
    return e, efo


def _tc_node(x, pa, pb, wx, wa, b1, w2, b2, w3, b3, lng, lnb, next_w1=None):
    base_specs = [_rows(BN), _rows(BN), _rows(BN), _rows(BN), _rows(BN),
                  _full((D, D)), _full((D, D)), _full((1, D)), _full((D, D)),
                  _full((1, D)), _full((D, D)), _full((1, D)), _full((1, D)),
                  _full((1, D))]
    args = (x, pa[:N], pa[NP:NP + N], pb[:N], pb[NP:NP + N],
            wx, wa, b1, w2, b2, w3, b3, lng, lnb)
    if next_w1 is None:
        return pl.pallas_call(
            _node_body,
            grid=(N // BN,),
            in_specs=base_specs,
            out_specs=_rows(BN),
            out_shape=jax.ShapeDtypeStruct((N, D), jnp.float32),
        )(*args), None, None
    return pl.pallas_call(
        _node_body_proj,
        grid=(N // BN,),
        in_specs=base_specs + [_full((D, D)), _full((D, D))],
        out_specs=[_rows(BN), _rows(BN), _rows(BN)],
        out_shape=[jax.ShapeDtypeStruct((N, D), jnp.float32)] * 3,
    )(*args, next_w1[:D], next_w1[D:2 * D])


def _row2(b):
    return jnp.reshape(b, (1, D))


def kernel(x, edge_index, edge_features, params):
    src = edge_index[0].astype(jnp.int32)
    dst = edge_index[1].astype(jnp.int32)
    src_h = (src[:E2], src[E2:])
    dst_h = (dst[:E2], dst[E2:])
    ef_h = (edge_features, edge_features)
    ef_off = (0, E2 // BE)
    nsteps = len(params)
    u = v = None
    for si, p in enumerate(params):
        (w1, b1), (w2, b2), (w3, b3) = p["edge_mlp"]
        lng_e, lnb_e = p["edge_ln"]
        (nw1, nb1), (nw2, nb2), (nw3, nb3) = p["node_mlp"]
        lng_n, lnb_n = p["node_ln"]
        last = si == nsteps - 1

        if u is None:
            u, v = _tc_proj(x, w1[:D], w1[D:2 * D])
        g0 = _sc_gather_half(u, v, src_h[0], dst_h[0])
        g1 = _sc_gather_half(u, v, src_h[1], dst_h[1])
        wef_bf = w1[2 * D:].astype(jnp.bfloat16)
        w2_bf = w2.astype(jnp.bfloat16)
        w3_bf = w3.astype(jnp.bfloat16)
        e0, efo0 = _tc_edge(g0, ef_h[0], wef_bf, _row2(b1), w2_bf, _row2(b2),
                            w3_bf, _row2(b3), _row2(lng_e), _row2(lnb_e),
                            ef_off[0], last)
        p0 = _sc_scatter_half(e0, dst_h[0])
        e1, efo1 = _tc_edge(g1, ef_h[1], wef_bf, _row2(b1), w2_bf, _row2(b2),
                            w3_bf, _row2(b3), _row2(lng_e), _row2(lnb_e),
                            ef_off[1], last)
        p1 = _sc_scatter_half(e1, dst_h[1])
        next_w1 = None if last else params[si + 1]["edge_mlp"][0][0]
        x, u, v = _tc_node(x, p0, p1, nw1[:D], nw1[D:], _row2(nb1), nw2,
                           _row2(nb2), nw3, _row2(nb3), _row2(lng_n),
                           _row2(lnb_n), next_w1)
        ef_h = (efo0, efo1)
        ef_off = (0, 0)
    return x
